# Initial kernel scaffold; baseline (speedup 1.0000x reference)
import functools

import jax
import jax.numpy as jnp
from jax import lax
from jax.experimental import pallas as pl
from jax.experimental.pallas import tpu as pltpu
from jax.experimental.pallas import tpu_sc as plsc

N = 10000
E = 320000
HID = 128
W = 144
TAIL = 16

NC, NS = 2, 16
NW = NC * NS
EPW = E // NW
KCH = 80
NCH = EPW // KCH
NPT = N // NS

F32 = jnp.float32


def _silu(x):
    return x * (1.0 / (1.0 + jnp.exp(-x)))


def _node0_body(h, cp, wemb, bemb, w1a, be1, w1b, hh_o, ta_o, tb_o):
    hh = jnp.dot(h[...], wemb[...], preferred_element_type=F32) + bemb[...]
    hh_o[...] = hh
    ha = jnp.dot(hh, w1a[...], preferred_element_type=F32) + be1[...]
    hb = jnp.dot(hh, w1b[...], preferred_element_type=F32)
    ta_o[...] = jnp.concatenate([ha, cp[...]], axis=1)
    tb_o[...] = jnp.concatenate([hb, -cp[...]], axis=1)


def _node_body(h, cp, vp, p0, p1, wv1, bv1, wv2, bv2, wn1a, wn1b, bn1, wn2,
               bn2, w1a, be1, w1b, hn_o, cn_o, ta_o, tb_o):
    hv = h[...]
    aggm = p0[:, :HID] + p1[:, :HID]
    tail = p0[:, HID:] + p1[:, HID:]
    lane = lax.broadcasted_iota(jnp.int32, (1, TAIL), 1)
    mask3 = (lane < 3).astype(F32)
    cnt = jnp.maximum(tail[:, 3:4], 1.0)
    sv = (jnp.dot(_silu(jnp.dot(hv, wv1[...], preferred_element_type=F32)
                        + bv1[...]), wv2[...], preferred_element_type=F32)
          + bv2[...])
    cn = cp[...] + (tail * mask3) / cnt + sv * vp[...]
    cn_o[...] = cn
    t = _silu(jnp.dot(hv, wn1a[...], preferred_element_type=F32)
              + jnp.dot(aggm, wn1b[...], preferred_element_type=F32)
              + bn1[...])
    hn = hv + jnp.dot(t, wn2[...], preferred_element_type=F32) + bn2[...]
    hn_o[...] = hn
    ha = jnp.dot(hn, w1a[...], preferred_element_type=F32) + be1[...]
    hb = jnp.dot(hn, w1b[...], preferred_element_type=F32)
    ta_o[...] = jnp.concatenate([ha, cn], axis=1)
    tb_o[...] = jnp.concatenate([hb, -cn], axis=1)


def _node_last_body(h, cp, vp, p0, p1, wv1, bv1, wv2, bv2, cn_o):
    hv = h[...]
    tail = p0[:, HID:] + p1[:, HID:]
    lane = lax.broadcasted_iota(jnp.int32, (1, TAIL), 1)
    mask3 = (lane < 3).astype(F32)
    cnt = jnp.maximum(tail[:, 3:4], 1.0)
    sv = (jnp.dot(_silu(jnp.dot(hv, wv1[...], preferred_element_type=F32)
                        + bv1[...]), wv2[...], preferred_element_type=F32)
          + bv2[...])
    cn_o[...] = cp[...] + (tail * mask3) / cnt + sv * vp[...]


def _edge_body(ga, gb, ea, wr, w1e, we2, be2, wc1, bc1, wc2, m_o):
    gh = ga[:, :HID] + gb[:, :HID]
    tail = ga[:, HID:] + gb[:, HID:]
    radial = jnp.sum(tail * tail, axis=1, keepdims=True)
    e1 = (gh + radial * wr[...]
          + jnp.dot(ea[...], w1e[...], preferred_element_type=F32))
    m = _silu(jnp.dot(_silu(e1), we2[...], preferred_element_type=F32)
              + be2[...])
    cm = _silu(jnp.dot(m, wc1[...], preferred_element_type=F32) + bc1[...])
    s = jnp.dot(cm, wc2[...], preferred_element_type=F32)
    lane = lax.broadcasted_iota(jnp.int32, (1, TAIL), 1)
    cvec = (lane == 3).astype(F32)
    m_o[...] = jnp.concatenate([m, tail * s + cvec], axis=1)


def _sc_gather_body(ta, tb, row, col, ga_o, gb_o, rowv, colv, bufa, bufb,
                    sema, semb):
    wid = lax.axis_index("s") * NC + lax.axis_index("c")
    base = wid * EPW

    def chunk(c, _):
        off = base + c * KCH
        pltpu.sync_copy(row.at[pl.ds(off, KCH)], rowv)
        pltpu.sync_copy(col.at[pl.ds(off, KCH)], colv)
        cpa = pltpu.async_copy(ta.at[rowv], bufa, sema)
        cpb = pltpu.async_copy(tb.at[colv], bufb, semb)
        cpa.wait()
        cpb.wait()
        pltpu.sync_copy(bufa, ga_o.at[pl.ds(off, KCH)])
        pltpu.sync_copy(bufb, gb_o.at[pl.ds(off, KCH)])
        return 0

    lax.fori_loop(0, NCH, chunk, 0)


def _sc_scatter_body(m, row, zrows, out, idxv, buf, acc, sem):
    cid = lax.axis_index("c")
    sid = lax.axis_index("s")
    wid = sid * NC + cid
    pltpu.sync_copy(zrows, acc.at[pl.ds(sid * NPT, NPT)])
    plsc.subcore_barrier()
    base = wid * EPW

    def chunk(c, _):
        off = base + c * KCH
        pltpu.sync_copy(row.at[pl.ds(off, KCH)], idxv)
        pltpu.sync_copy(m.at[pl.ds(off, KCH)], buf)
        pltpu.sync_copy(buf, acc.at[idxv], add=True)
        return 0

    lax.fori_loop(0, NCH, chunk, 0)
    plsc.subcore_barrier()
    pltpu.sync_copy(acc.at[pl.ds(sid * NPT, NPT)],
                    out.at[cid, pl.ds(sid * NPT, NPT)])


_MESH = plsc.VectorSubcoreMesh(core_axis_name="c", subcore_axis_name="s",
                               num_cores=NC, num_subcores=NS)

_sc_gather = functools.partial(
    pl.kernel,
    out_type=(jax.ShapeDtypeStruct((E, W), F32),
              jax.ShapeDtypeStruct((E, W), F32)),
    mesh=_MESH,
    scratch_types=[
        pltpu.VMEM((KCH,), jnp.int32),
        pltpu.VMEM((KCH,), jnp.int32),
        pltpu.VMEM((KCH, W), F32),
        pltpu.VMEM((KCH, W), F32),
        pltpu.SemaphoreType.DMA,
        pltpu.SemaphoreType.DMA,
    ],
)(_sc_gather_body)

_sc_scatter = functools.partial(
    pl.kernel,
    out_type=jax.ShapeDtypeStruct((NC, N, W), F32),
    mesh=_MESH,
    scratch_types=[
        pltpu.VMEM((KCH,), jnp.int32),
        pltpu.VMEM((KCH, W), F32),
        pltpu.VMEM_SHARED((N, W), F32),
        pltpu.SemaphoreType.DMA,
    ],
)(_sc_scatter_body)

BN = 2000
BE = 2000


def _full(shape):
    return pl.BlockSpec(shape, lambda i: (0,) * len(shape))


def _blk(shape):
    return pl.BlockSpec(shape, lambda i: (i,) + (0,) * (len(shape) - 1))


def _tc_node0(h, cp, wemb, bemb, w1a, be1, w1b):
    return pl.pallas_call(
        _node0_body,
        grid=(N // BN,),
        in_specs=[_blk((BN, HID)), _blk((BN, TAIL)), _full((HID, HID)),
                  _full((1, HID)), _full((HID, HID)), _full((1, HID)),
                  _full((HID, HID))],
        out_specs=[_blk((BN, HID)), _blk((BN, W)), _blk((BN, W))],
        out_shape=[jax.ShapeDtypeStruct((N, HID), F32),
                   jax.ShapeDtypeStruct((N, W), F32),
                   jax.ShapeDtypeStruct((N, W), F32)],
    )(h, cp, wemb, bemb, w1a, be1, w1b)


def _tc_node(h, cp, vp, p0, p1, wv1, bv1, wv2, bv2, wn1a, wn1b, bn1, wn2,
             bn2, w1a, be1, w1b):
    return pl.pallas_call(
        _node_body,
        grid=(N // BN,),
        in_specs=[_blk((BN, HID)), _blk((BN, TAIL)), _blk((BN, TAIL)),
                  _blk((BN, W)), _blk((BN, W)),
                  _full((HID, HID)), _full((1, HID)), _full((HID, 1)),
                  _full((1, 1)),
                  _full((HID, HID)), _full((HID, HID)), _full((1, HID)),
                  _full((HID, HID)), _full((1, HID)),
                  _full((HID, HID)), _full((1, HID)), _full((HID, HID))],
        out_specs=[_blk((BN, HID)), _blk((BN, TAIL)), _blk((BN, W)),
                   _blk((BN, W))],
        out_shape=[jax.ShapeDtypeStruct((N, HID), F32),
                   jax.ShapeDtypeStruct((N, TAIL), F32),
                   jax.ShapeDtypeStruct((N, W), F32),
                   jax.ShapeDtypeStruct((N, W), F32)],
    )(h, cp, vp, p0, p1, wv1, bv1, wv2, bv2, wn1a, wn1b, bn1, wn2, bn2,
      w1a, be1, w1b)


def _tc_node_last(h, cp, vp, p0, p1, wv1, bv1, wv2, bv2):
    return pl.pallas_call(
        _node_last_body,
        grid=(N // BN,),
        in_specs=[_blk((BN, HID)), _blk((BN, TAIL)), _blk((BN, TAIL)),
                  _blk((BN, W)), _blk((BN, W)),
                  _full((HID, HID)), _full((1, HID)), _full((HID, 1)),
                  _full((1, 1))],
        out_specs=_blk((BN, TAIL)),
        out_shape=jax.ShapeDtypeStruct((N, TAIL), F32),
    )(h, cp, vp, p0, p1, wv1, bv1, wv2, bv2)


def _tc_edge(ga, gb, ea, wr, w1e, we2, be2, wc1, bc1, wc2):
    return pl.pallas_call(
        _edge_body,
        grid=(E // BE,),
        in_specs=[_blk((BE, W)), _blk((BE, W)), _blk((BE, 16)),
                  _full((1, HID)), _full((16, HID)), _full((HID, HID)),
                  _full((1, HID)), _full((HID, HID)), _full((1, HID)),
                  _full((HID, 1))],
        out_specs=_blk((BE, W)),
        out_shape=jax.ShapeDtypeStruct((E, W), F32),
    )(ga, gb, ea, wr, w1e, we2, be2, wc1, bc1, wc2)


def kernel(h, x, edges, vel, edge_attr, params):
    row = edges[0]
    col = edges[1]
    cp = jnp.pad(x, ((0, 0), (0, TAIL - 3)))
    vp = jnp.pad(vel, ((0, 0), (0, TAIL - 3)))
    zrows = jnp.zeros((NPT, W), F32)
    r2 = lambda b: b.reshape(1, -1)

    lp = params["layers"]
    p0w = lp[0]
    hh, ta, tb = _tc_node0(
        h, cp, params["emb"]["W"], r2(params["emb"]["b"]),
        p0w["We1"][:HID], r2(p0w["be1"]), p0w["We1"][HID:2 * HID])

    for li in range(4):
        p = lp[li]
        ga, gb = _sc_gather(ta, tb, row, col)
        m = _tc_edge(ga, gb, edge_attr, r2(p["We1"][2 * HID]),
                     p["We1"][2 * HID + 1:], p["We2"], r2(p["be2"]),
                     p["Wc1"], r2(p["bc1"]), p["Wc2"])
        part = _sc_scatter(m, row, zrows)
        if li < 3:
            nx = lp[li + 1]
            hh, cp, ta, tb = _tc_node(
                hh, cp, vp, part[0], part[1],
                p["Wv1"], r2(p["bv1"]), p["Wv2"], r2(p["bv2"]),
                p["Wn1"][:HID], p["Wn1"][HID:], r2(p["bn1"]),
                p["Wn2"], r2(p["bn2"]),
                nx["We1"][:HID], r2(nx["be1"]), nx["We1"][HID:2 * HID])
        else:
            cp = _tc_node_last(hh, cp, vp, part[0], part[1],
                               p["Wv1"], r2(p["bv1"]), p["Wv2"],
                               r2(p["bv2"]))
    return cp[:, :3]


# speedup vs baseline: 2.8759x; 2.8759x over previous
import functools

import jax
import jax.numpy as jnp
from jax import lax
from jax.experimental import pallas as pl
from jax.experimental.pallas import tpu as pltpu
from jax.experimental.pallas import tpu_sc as plsc

N = 10000
E = 320000
HID = 128
TAIL = 16

NC, NS = 2, 16
NW = NC * NS
EPW = E // NW
KCH = 80
NCH = EPW // KCH
NP = 10240
SPT = NP // NS
NT = 8
TR = (N * NT) // HID

F32 = jnp.float32


def _silu(x):
    return x * (1.0 / (1.0 + jnp.exp(-x)))


def _node0_body(h, cp, wemb, bemb, w1a, be1, w1b, hh_o, ta_o, tb_o):
    hh = jnp.dot(h[...], wemb[...], preferred_element_type=F32) + bemb[...]
    hh_o[...] = hh
    ta_o[...] = jnp.dot(hh, w1a[...], preferred_element_type=F32) + be1[...]
    tb_o[...] = jnp.dot(hh, w1b[...], preferred_element_type=F32)


def _node_body(h, cp, vp, p0, p1, pt, wv1, bv1, wv2, bv2, wn1a, wn1b, bn1,
               wn2, bn2, w1a, be1, w1b, hn_o, cn_o, ta_o, tb_o):
    hv = h[...]
    aggm = p0[...] + p1[...]
    tl = pt[...]
    cnt = jnp.maximum(tl[:, 3:4], 1.0)
    lane = lax.broadcasted_iota(jnp.int32, (1, TAIL), 1)
    mask3 = (lane < 3).astype(F32)
    tl16 = jnp.concatenate([tl, jnp.zeros_like(tl)], axis=1)
    sv = (jnp.dot(_silu(jnp.dot(hv, wv1[...], preferred_element_type=F32)
                        + bv1[...]), wv2[...], preferred_element_type=F32)
          + bv2[...])
    cn = cp[...] + (tl16 * mask3) / cnt + sv * vp[...]
    cn_o[...] = cn
    t = _silu(jnp.dot(hv, wn1a[...], preferred_element_type=F32)
              + jnp.dot(aggm, wn1b[...], preferred_element_type=F32)
              + bn1[...])
    hn = hv + jnp.dot(t, wn2[...], preferred_element_type=F32) + bn2[...]
    hn_o[...] = hn
    ta_o[...] = jnp.dot(hn, w1a[...], preferred_element_type=F32) + be1[...]
    tb_o[...] = jnp.dot(hn, w1b[...], preferred_element_type=F32)


def _node_last_body(h, cp, vp, pt, wv1, bv1, wv2, bv2, cn_o):
    hv = h[...]
    tl = pt[...]
    cnt = jnp.maximum(tl[:, 3:4], 1.0)
    lane = lax.broadcasted_iota(jnp.int32, (1, TAIL), 1)
    mask3 = (lane < 3).astype(F32)
    tl16 = jnp.concatenate([tl, jnp.zeros_like(tl)], axis=1)
    sv = (jnp.dot(_silu(jnp.dot(hv, wv1[...], preferred_element_type=F32)
                        + bv1[...]), wv2[...], preferred_element_type=F32)
          + bv2[...])
    cn_o[...] = cp[...] + (tl16 * mask3) / cnt + sv * vp[...]


def _edge_body(ga, gb, es, ea, wr, w1e, we2, be2, wc1, bc1, wc2, m_o, s_o):
    radial = es[:, 0:1]
    e1 = (ga[...] + gb[...] + radial * wr[...]
          + jnp.dot(ea[...], w1e[...], preferred_element_type=F32))
    m = _silu(jnp.dot(_silu(e1), we2[...], preferred_element_type=F32)
              + be2[...])
    m_o[...] = m
    cm = _silu(jnp.dot(m, wc1[...], preferred_element_type=F32) + bc1[...])
    s = jnp.dot(cm, wc2[...], preferred_element_type=F32)
    s_o[...] = jnp.concatenate(
        [s, jnp.zeros((s.shape[0], TAIL - 1), F32)], axis=1)


def _reduce_body(pin, out):
    out[...] = jnp.sum(pin[...], axis=0)


def _sc_gather_body(ta, tb, row, col, cx, cy, cz, ga_o, gb_o, es_o,
                    rowv, colv, bufa, bufb, esb, cxv, cyv, czv, sema, semb):
    wid = lax.axis_index("s") * NC + lax.axis_index("c")
    base = wid * EPW
    pltpu.sync_copy(cx, cxv)
    pltpu.sync_copy(cy, cyv)
    pltpu.sync_copy(cz, czv)
    iota = lax.iota(jnp.int32, 16)

    def chunk(ci, _):
        off = base + ci * KCH
        pltpu.sync_copy(row.at[pl.ds(off, KCH)], rowv)
        pltpu.sync_copy(col.at[pl.ds(off, KCH)], colv)
        cpa = pltpu.async_copy(ta.at[rowv], bufa, sema)
        cpb = pltpu.async_copy(tb.at[colv], bufb, semb)
        for j in range(KCH // 16):
            rv = rowv[pl.ds(j * 16, 16)]
            cv = colv[pl.ds(j * 16, 16)]
            dx = plsc.load_gather(cxv, [rv]) - plsc.load_gather(cxv, [cv])
            dy = plsc.load_gather(cyv, [rv]) - plsc.load_gather(cyv, [cv])
            dz = plsc.load_gather(czv, [rv]) - plsc.load_gather(czv, [cv])
            r = dx * dx + dy * dy + dz * dz
            ri = iota + (j * 16)
            plsc.store_scatter(esb, [ri, iota * 0], r)
            plsc.store_scatter(esb, [ri, iota * 0 + 1], dx)
            plsc.store_scatter(esb, [ri, iota * 0 + 2], dy)
            plsc.store_scatter(esb, [ri, iota * 0 + 3], dz)
        cpa.wait()
        cpb.wait()
        pltpu.sync_copy(bufa, ga_o.at[pl.ds(off, KCH)])
        pltpu.sync_copy(bufb, gb_o.at[pl.ds(off, KCH)])
        pltpu.sync_copy(esb, es_o.at[pl.ds(off, KCH)])
        return 0

    lax.fori_loop(0, NCH, chunk, 0)


def _sc_scatter_m_body(m, row, zrows, pm_o, idr, mb, accm, sem):
    cid = lax.axis_index("c")
    sid = lax.axis_index("s")
    wid = sid * NC + cid
    base = wid * EPW
    pltpu.sync_copy(zrows, accm.at[pl.ds(sid * SPT, SPT)])
    plsc.subcore_barrier()

    def chunk(ci, _):
        off = base + ci * KCH
        pltpu.sync_copy(row.at[pl.ds(off, KCH)], idr)
        pltpu.sync_copy(m.at[pl.ds(off, KCH)], mb)
        pltpu.sync_copy(mb, accm.at[idr], add=True)
        return 0

    lax.fori_loop(0, NCH, chunk, 0)
    plsc.subcore_barrier()
    pltpu.sync_copy(accm.at[pl.ds(sid * SPT, SPT)],
                    pm_o.at[cid, pl.ds(sid * SPT, SPT)])


def _sc_scatter_t_body(s16, es, row, pt_o, idr, sb, esb, acct):
    cid = lax.axis_index("c")
    sid = lax.axis_index("s")
    wid = sid * NC + cid
    base = wid * EPW
    zero16 = jnp.zeros((16,), F32)

    def zbody(i, _):
        for j in range(HID // 16):
            acct[i, pl.ds(j * 16, 16)] = zero16
        return 0

    lax.fori_loop(0, TR, zbody, 0)
    iota = lax.iota(jnp.int32, 16)
    one16 = zero16 + 1.0

    def chunk(ci, _):
        off = base + ci * KCH
        pltpu.sync_copy(row.at[pl.ds(off, KCH)], idr)
        pltpu.sync_copy(s16.at[pl.ds(off, KCH)], sb)
        pltpu.sync_copy(es.at[pl.ds(off, KCH)], esb)
        for j in range(KCH // 16):
            rv = idr[pl.ds(j * 16, 16)]
            ri = iota + (j * 16)
            sv = plsc.load_gather(sb, [ri, iota * 0])
            dx = plsc.load_gather(esb, [ri, iota * 0 + 1])
            dy = plsc.load_gather(esb, [ri, iota * 0 + 2])
            dz = plsc.load_gather(esb, [ri, iota * 0 + 3])
            fb = rv * NT
            for k, v in ((0, dx * sv), (1, dy * sv), (2, dz * sv),
                         (3, one16)):
                fk = fb + k
                plsc.addupdate_scatter(
                    acct, [lax.shift_right_logical(fk, 7), fk & 127], v)
        return 0

    lax.fori_loop(0, NCH, chunk, 0)
    pltpu.sync_copy(acct, pt_o.at[wid])


@functools.cache
def _sc_kernels():
    mesh = plsc.VectorSubcoreMesh(core_axis_name="c", subcore_axis_name="s",
                                  num_cores=NC, num_subcores=NS)
    cparams = pltpu.CompilerParams(needs_layout_passes=False)
    gather = functools.partial(
        pl.kernel,
        compiler_params=cparams,
        out_type=(jax.ShapeDtypeStruct((E, HID), F32),
                  jax.ShapeDtypeStruct((E, HID), F32),
                  jax.ShapeDtypeStruct((E, TAIL), F32)),
        mesh=mesh,
        scratch_types=[
            pltpu.VMEM((KCH,), jnp.int32),
            pltpu.VMEM((KCH,), jnp.int32),
            pltpu.VMEM((KCH, HID), F32),
            pltpu.VMEM((KCH, HID), F32),
            pltpu.VMEM((KCH, TAIL), F32),
            pltpu.VMEM((N,), F32),
            pltpu.VMEM((N,), F32),
            pltpu.VMEM((N,), F32),
            pltpu.SemaphoreType.DMA,
            pltpu.SemaphoreType.DMA,
        ],
    )(_sc_gather_body)
    scatter_m = functools.partial(
        pl.kernel,
        compiler_params=cparams,
        out_type=jax.ShapeDtypeStruct((NC, NP, HID), F32),
        mesh=mesh,
        scratch_types=[
            pltpu.VMEM((KCH,), jnp.int32),
            pltpu.VMEM((KCH, HID), F32),
            pltpu.VMEM_SHARED((NP, HID), F32),
            pltpu.SemaphoreType.DMA,
        ],
    )(_sc_scatter_m_body)
    scatter_t = functools.partial(
        pl.kernel,
        compiler_params=cparams,
        out_type=jax.ShapeDtypeStruct((NW, TR, HID), F32),
        mesh=mesh,
        scratch_types=[
            pltpu.VMEM((KCH,), jnp.int32),
            pltpu.VMEM((KCH, TAIL), F32),
            pltpu.VMEM((KCH, TAIL), F32),
            pltpu.VMEM((TR, HID), F32),
        ],
    )(_sc_scatter_t_body)
    return gather, scatter_m, scatter_t


def _sc_gather(ta, tb, row, col, cx, cy, cz):
    return _sc_kernels()[0](ta, tb, row, col, cx, cy, cz)


def _sc_scatter_m(m, row, zrows):
    return _sc_kernels()[1](m, row, zrows)


def _sc_scatter_t(s16, es, row):
    return _sc_kernels()[2](s16, es, row)


BN = 2000
BE = 2000


def _full(shape):
    return pl.BlockSpec(shape, lambda i: (0,) * len(shape))


def _blk(shape, pos=0):
    def imap(i):
        out = [0] * len(shape)
        out[pos] = i
        return tuple(out)
    return pl.BlockSpec(shape, imap)


def _tc_node0(h, cp, wemb, bemb, w1a, be1, w1b):
    return pl.pallas_call(
        _node0_body,
        grid=(N // BN,),
        in_specs=[_blk((BN, HID)), _blk((BN, TAIL)), _full((HID, HID)),
                  _full((1, HID)), _full((HID, HID)), _full((1, HID)),
                  _full((HID, HID))],
        out_specs=[_blk((BN, HID)), _blk((BN, HID)), _blk((BN, HID))],
        out_shape=[jax.ShapeDtypeStruct((N, HID), F32),
                   jax.ShapeDtypeStruct((N, HID), F32),
                   jax.ShapeDtypeStruct((N, HID), F32)],
    )(h, cp, wemb, bemb, w1a, be1, w1b)


def _tc_node(h, cp, vp, p0, p1, pt, wv1, bv1, wv2, bv2, wn1a, wn1b, bn1,
             wn2, bn2, w1a, be1, w1b):
    return pl.pallas_call(
        _node_body,
        grid=(N // BN,),
        in_specs=[_blk((BN, HID)), _blk((BN, TAIL)), _blk((BN, TAIL)),
                  _blk((BN, HID)), _blk((BN, HID)), _blk((BN, NT)),
                  _full((HID, HID)), _full((1, HID)), _full((HID, 1)),
                  _full((1, 1)),
                  _full((HID, HID)), _full((HID, HID)), _full((1, HID)),
                  _full((HID, HID)), _full((1, HID)),
                  _full((HID, HID)), _full((1, HID)), _full((HID, HID))],
        out_specs=[_blk((BN, HID)), _blk((BN, TAIL)), _blk((BN, HID)),
                   _blk((BN, HID))],
        out_shape=[jax.ShapeDtypeStruct((N, HID), F32),
                   jax.ShapeDtypeStruct((N, TAIL), F32),
                   jax.ShapeDtypeStruct((N, HID), F32),
                   jax.ShapeDtypeStruct((N, HID), F32)],
    )(h, cp, vp, p0, p1, pt, wv1, bv1, wv2, bv2, wn1a, wn1b, bn1, wn2, bn2,
      w1a, be1, w1b)


def _tc_node_last(h, cp, vp, pt, wv1, bv1, wv2, bv2):
    return pl.pallas_call(
        _node_last_body,
        grid=(N // BN,),
        in_specs=[_blk((BN, HID)), _blk((BN, TAIL)), _blk((BN, TAIL)),
                  _blk((BN, NT)),
                  _full((HID, HID)), _full((1, HID)), _full((HID, 1)),
                  _full((1, 1))],
        out_specs=_blk((BN, TAIL)),
        out_shape=jax.ShapeDtypeStruct((N, TAIL), F32),
    )(h, cp, vp, pt, wv1, bv1, wv2, bv2)


def _tc_reduce(pt):
    return pl.pallas_call(
        _reduce_body,
        grid=(1,),
        in_specs=[_full((NW, TR, HID))],
        out_specs=_full((TR, HID)),
        out_shape=jax.ShapeDtypeStruct((TR, HID), F32),
    )(pt)


def _tc_edge(ga, gb, es, ea, wr, w1e, we2, be2, wc1, bc1, wc2):
    return pl.pallas_call(
        _edge_body,
        grid=(E // BE,),
        in_specs=[_blk((BE, HID)), _blk((BE, HID)), _blk((BE, TAIL)),
                  _blk((BE, 16)),
                  _full((1, HID)), _full((16, HID)), _full((HID, HID)),
                  _full((1, HID)), _full((HID, HID)), _full((1, HID)),
                  _full((HID, 1))],
        out_specs=[_blk((BE, HID)), _blk((BE, TAIL))],
        out_shape=[jax.ShapeDtypeStruct((E, HID), F32),
                   jax.ShapeDtypeStruct((E, TAIL), F32)],
    )(ga, gb, es, ea, wr, w1e, we2, be2, wc1, bc1, wc2)


def kernel(h, x, edges, vel, edge_attr, params):
    row = edges[0]
    col = edges[1]
    cp = jnp.pad(x, ((0, 0), (0, TAIL - 3)))
    vp = jnp.pad(vel, ((0, 0), (0, TAIL - 3)))
    zrows = jnp.zeros((SPT, HID), F32)
    r2 = lambda b: b.reshape(1, -1)

    lp = params["layers"]
    p0w = lp[0]
    hh, ta, tb = _tc_node0(
        h, cp, params["emb"]["W"], r2(params["emb"]["b"]),
        p0w["We1"][:HID], r2(p0w["be1"]), p0w["We1"][HID:2 * HID])

    for li in range(4):
        p = lp[li]
        cx, cy, cz = cp[:, 0], cp[:, 1], cp[:, 2]
        ga, gb, es = _sc_gather(ta, tb, row, col, cx, cy, cz)
        m, s16 = _tc_edge(ga, gb, es, edge_attr, r2(p["We1"][2 * HID]),
                          p["We1"][2 * HID + 1:], p["We2"], r2(p["be2"]),
                          p["Wc1"], r2(p["bc1"]), p["Wc2"])
        pm = _sc_scatter_m(m, row, zrows)
        pt = _sc_scatter_t(s16, es, row)
        ptr = _tc_reduce(pt).reshape(N, NT)
        if li < 3:
            nx = lp[li + 1]
            hh, cp, ta, tb = _tc_node(
                hh, cp, vp, pm[0, :N], pm[1, :N], ptr,
                p["Wv1"], r2(p["bv1"]), p["Wv2"], r2(p["bv2"]),
                p["Wn1"][:HID], p["Wn1"][HID:], r2(p["bn1"]),
                p["Wn2"], r2(p["bn2"]),
                nx["We1"][:HID], r2(nx["be1"]), nx["We1"][HID:2 * HID])
        else:
            cp = _tc_node_last(hh, cp, vp, ptr,
                               p["Wv1"], r2(p["bv1"]), p["Wv2"],
                               r2(p["bv2"]))
    return cp[:, :3]


# speedup vs baseline: 4.2636x; 1.4826x over previous
import functools

import jax
import jax.numpy as jnp
from jax import lax
from jax.experimental import pallas as pl
from jax.experimental.pallas import tpu as pltpu
from jax.experimental.pallas import tpu_sc as plsc

N = 10000
E = 320000
HID = 128
TAIL = 16

NC, NS = 2, 16
NW = NC * NS
EPW = E // NW
KCH = 80
NCH = EPW // KCH
NP = 10240
SPT = NP // NS
NT = 4
TR = 320

F32 = jnp.float32


def _silu(x):
    return x * (1.0 / (1.0 + jnp.exp(-x)))


def _node0_body(h, cp, wemb, bemb, w1a, be1, w1b, hh_o, ta_o, tb_o):
    hh = jnp.dot(h[...], wemb[...], preferred_element_type=F32) + bemb[...]
    hh_o[...] = hh
    ta_o[...] = jnp.dot(hh, w1a[...], preferred_element_type=F32) + be1[...]
    tb_o[...] = jnp.dot(hh, w1b[...], preferred_element_type=F32)


def _node_body(h, cp, vp, p0, p1, pt, wv1, bv1, wv2, bv2, wn1a, wn1b, bn1,
               wn2, bn2, w1a, be1, w1b, hn_o, cn_o, ta_o, tb_o):
    hv = h[...]
    aggm = p0[...] + p1[...]
    tl = pt[...]
    cnt = jnp.maximum(tl[:, 3:4], 1.0)
    lane = lax.broadcasted_iota(jnp.int32, (1, TAIL), 1)
    mask3 = (lane < 3).astype(F32)
    tl16 = jnp.concatenate(
        [tl, jnp.zeros((tl.shape[0], TAIL - NT), F32)], axis=1)
    sv = (jnp.dot(_silu(jnp.dot(hv, wv1[...], preferred_element_type=F32)
                        + bv1[...]), wv2[...], preferred_element_type=F32)
          + bv2[...])
    cn = cp[...] + (tl16 * mask3) / cnt + sv * vp[...]
    cn_o[...] = cn
    t = _silu(jnp.dot(hv, wn1a[...], preferred_element_type=F32)
              + jnp.dot(aggm, wn1b[...], preferred_element_type=F32)
              + bn1[...])
    hn = hv + jnp.dot(t, wn2[...], preferred_element_type=F32) + bn2[...]
    hn_o[...] = hn
    ta_o[...] = jnp.dot(hn, w1a[...], preferred_element_type=F32) + be1[...]
    tb_o[...] = jnp.dot(hn, w1b[...], preferred_element_type=F32)


def _node_last_body(h, cp, vp, pt, wv1, bv1, wv2, bv2, cn_o):
    hv = h[...]
    tl = pt[...]
    cnt = jnp.maximum(tl[:, 3:4], 1.0)
    lane = lax.broadcasted_iota(jnp.int32, (1, TAIL), 1)
    mask3 = (lane < 3).astype(F32)
    tl16 = jnp.concatenate(
        [tl, jnp.zeros((tl.shape[0], TAIL - NT), F32)], axis=1)
    sv = (jnp.dot(_silu(jnp.dot(hv, wv1[...], preferred_element_type=F32)
                        + bv1[...]), wv2[...], preferred_element_type=F32)
          + bv2[...])
    cn_o[...] = cp[...] + (tl16 * mask3) / cnt + sv * vp[...]


def _edge_body(ga, gb, es, ea, wr, w1e, we2, be2, wc1, bc1, wc2, m_o, s_o):
    radial = es[:, 0:1]
    e1 = (ga[...] + gb[...] + radial * wr[...]
          + jnp.dot(ea[...], w1e[...], preferred_element_type=F32))
    m = _silu(jnp.dot(_silu(e1), we2[...], preferred_element_type=F32)
              + be2[...])
    m_o[...] = m
    cm = _silu(jnp.dot(m, wc1[...], preferred_element_type=F32) + bc1[...])
    s = jnp.dot(cm, wc2[...], preferred_element_type=F32)
    s_o[...] = jnp.concatenate(
        [s, jnp.zeros((s.shape[0], TAIL - 1), F32)], axis=1)


def _reduce_body(pin, out):
    out[...] = jnp.sum(pin[...], axis=0)


def _sc_gather_body(ta, tb, row3, col3, cx, cy, cz, ga_o, gb_o, es_o,
                    rowi, coli, bufa0, bufb0, bufa1, bufb1, esb,
                    cxv, cyv, czv, sga0, sgb0, sga1, sgb1, semw0, semw1):
    wid = lax.axis_index("s") * NC + lax.axis_index("c")
    base = wid * EPW
    pltpu.sync_copy(row3.at[wid], rowi)
    pltpu.sync_copy(col3.at[wid], coli)
    pltpu.sync_copy(cx, cxv)
    pltpu.sync_copy(cy, cyv)
    pltpu.sync_copy(cz, czv)
    iota = lax.iota(jnp.int32, 16)

    def issue_g(c, bufa, bufb, sa, sb):
        pltpu.async_copy(ta.at[rowi.at[c]], bufa, sa)
        pltpu.async_copy(tb.at[coli.at[c]], bufb, sb)

    def wait_g(c, bufa, bufb, sa, sb):
        pltpu.make_async_copy(ta.at[rowi.at[c]], bufa, sa).wait()
        pltpu.make_async_copy(tb.at[coli.at[c]], bufb, sb).wait()

    def compute_es(c, esb):
        for j in range(KCH // 16):
            rv = rowi[c, pl.ds(j * 16, 16)]
            cv = coli[c, pl.ds(j * 16, 16)]
            dx = plsc.load_gather(cxv, [rv]) - plsc.load_gather(cxv, [cv])
            dy = plsc.load_gather(cyv, [rv]) - plsc.load_gather(cyv, [cv])
            dz = plsc.load_gather(czv, [rv]) - plsc.load_gather(czv, [cv])
            r = dx * dx + dy * dy + dz * dz
            ri = iota + (j * 16)
            plsc.store_scatter(esb, [ri, iota * 0], r)
            plsc.store_scatter(esb, [ri, iota * 0 + 1], dx)
            plsc.store_scatter(esb, [ri, iota * 0 + 2], dy)
            plsc.store_scatter(esb, [ri, iota * 0 + 3], dz)

    def drain(c, bufa, bufb, esb, semw):
        off = base + c * KCH
        wa = pltpu.async_copy(bufa, ga_o.at[pl.ds(off, KCH)], semw)
        wb = pltpu.async_copy(bufb, gb_o.at[pl.ds(off, KCH)], semw)
        we = pltpu.async_copy(esb, es_o.at[pl.ds(off, KCH)], semw)
        wa.wait()
        wb.wait()
        we.wait()

    issue_g(0, bufa0, bufb0, sga0, sgb0)
    issue_g(1, bufa1, bufb1, sga1, sgb1)

    def body(i, _):
        c0 = 2 * i
        c1 = c0 + 1
        compute_es(c0, esb)
        wait_g(c0, bufa0, bufb0, sga0, sgb0)
        drain(c0, bufa0, bufb0, esb, semw0)
        issue_g(c0 + 2, bufa0, bufb0, sga0, sgb0)
        compute_es(c1, esb)
        wait_g(c1, bufa1, bufb1, sga1, sgb1)
        drain(c1, bufa1, bufb1, esb, semw1)

        @pl.when(i < (NCH - 1) // 2 - 1)
        def _():
            issue_g(c1 + 2, bufa1, bufb1, sga1, sgb1)

        return 0

    lax.fori_loop(0, (NCH - 1) // 2, body, 0)
    cl = NCH - 1
    compute_es(cl, esb)
    wait_g(cl, bufa0, bufb0, sga0, sgb0)
    drain(cl, bufa0, bufb0, esb, semw0)


def _sc_scatter_m_body(m, row3, zrows, pm_o, idr, mb0, mb1, accm,
                       sl0, sl1, sa0, sa1):
    cid = lax.axis_index("c")
    sid = lax.axis_index("s")
    wid = sid * NC + cid
    base = wid * EPW

    def issue_l(c, mb, sl):
        pltpu.async_copy(m.at[pl.ds(base + c * KCH, KCH)], mb, sl)

    def wait_l(c, mb, sl):
        pltpu.make_async_copy(m.at[pl.ds(base + c * KCH, KCH)], mb,
                              sl).wait()

    issue_l(0, mb0, sl0)
    issue_l(1, mb1, sl1)
    pltpu.sync_copy(row3.at[wid], idr)
    pltpu.sync_copy(zrows, accm.at[pl.ds(sid * SPT, SPT)])
    plsc.subcore_barrier()

    def sadd(c, mb, sa):
        pltpu.async_copy(mb, accm.at[idr.at[c]], sa, add=True).wait()

    def body(i, _):
        c0 = 2 * i
        c1 = c0 + 1
        wait_l(c0, mb0, sl0)
        sadd(c0, mb0, sa0)
        issue_l(c0 + 2, mb0, sl0)
        wait_l(c1, mb1, sl1)
        sadd(c1, mb1, sa1)

        @pl.when(i < (NCH - 1) // 2 - 1)
        def _():
            issue_l(c1 + 2, mb1, sl1)

        return 0

    lax.fori_loop(0, (NCH - 1) // 2, body, 0)
    cl = NCH - 1
    wait_l(cl, mb0, sl0)
    sadd(cl, mb0, sa0)
    plsc.subcore_barrier()
    pltpu.sync_copy(accm.at[pl.ds(sid * SPT, SPT)],
                    pm_o.at[cid, pl.ds(sid * SPT, SPT)])


def _sc_scatter_t_body(s16, es, row3, pt_o, idr, sb0, eb0, sb1, eb1, acct,
                       ss0, ss1):
    cid = lax.axis_index("c")
    sid = lax.axis_index("s")
    wid = sid * NC + cid
    base = wid * EPW

    def issue_l(c, sb, eb, ss):
        off = base + c * KCH
        pltpu.async_copy(s16.at[pl.ds(off, KCH)], sb, ss)
        pltpu.async_copy(es.at[pl.ds(off, KCH)], eb, ss)

    def wait_l(c, sb, eb, ss):
        off = base + c * KCH
        pltpu.make_async_copy(s16.at[pl.ds(off, KCH)], sb, ss).wait()
        pltpu.make_async_copy(es.at[pl.ds(off, KCH)], eb, ss).wait()

    issue_l(0, sb0, eb0, ss0)
    issue_l(1, sb1, eb1, ss1)
    pltpu.sync_copy(row3.at[wid], idr)
    zero16 = jnp.zeros((16,), F32)

    def zbody(i, _):
        for j in range(HID // 16):
            acct[i, pl.ds(j * 16, 16)] = zero16
        return 0

    lax.fori_loop(0, TR, zbody, 0)
    iota = lax.iota(jnp.int32, 16)
    one16 = zero16 + 1.0

    def process(c, sb, eb):
        for j in range(KCH // 16):
            rv = idr[c, pl.ds(j * 16, 16)]
            ri = iota + (j * 16)
            sv = plsc.load_gather(sb, [ri, iota * 0])
            dx = plsc.load_gather(eb, [ri, iota * 0 + 1])
            dy = plsc.load_gather(eb, [ri, iota * 0 + 2])
            dz = plsc.load_gather(eb, [ri, iota * 0 + 3])
            fb = rv * NT
            for k, v in ((0, dx * sv), (1, dy * sv), (2, dz * sv),
                         (3, one16)):
                fk = fb + k
                plsc.addupdate_scatter(
                    acct, [lax.shift_right_logical(fk, 7), fk & 127], v)

    def body(i, _):
        c0 = 2 * i
        c1 = c0 + 1
        wait_l(c0, sb0, eb0, ss0)
        process(c0, sb0, eb0)
        issue_l(c0 + 2, sb0, eb0, ss0)
        wait_l(c1, sb1, eb1, ss1)
        process(c1, sb1, eb1)

        @pl.when(i < (NCH - 1) // 2 - 1)
        def _():
            issue_l(c1 + 2, sb1, eb1, ss1)

        return 0

    lax.fori_loop(0, (NCH - 1) // 2, body, 0)
    cl = NCH - 1
    wait_l(cl, sb0, eb0, ss0)
    process(cl, sb0, eb0)
    pltpu.sync_copy(acct, pt_o.at[wid])


@functools.cache
def _sc_kernels():
    mesh = plsc.VectorSubcoreMesh(core_axis_name="c", subcore_axis_name="s",
                                  num_cores=NC, num_subcores=NS)
    cparams = pltpu.CompilerParams(needs_layout_passes=False)
    gather = functools.partial(
        pl.kernel,
        compiler_params=cparams,
        out_type=(jax.ShapeDtypeStruct((E, HID), F32),
                  jax.ShapeDtypeStruct((E, HID), F32),
                  jax.ShapeDtypeStruct((E, TAIL), F32)),
        mesh=mesh,
        scratch_types=[
            pltpu.VMEM((NCH, KCH), jnp.int32),
            pltpu.VMEM((NCH, KCH), jnp.int32),
            pltpu.VMEM((KCH, HID), F32),
            pltpu.VMEM((KCH, HID), F32),
            pltpu.VMEM((KCH, HID), F32),
            pltpu.VMEM((KCH, HID), F32),
            pltpu.VMEM((KCH, TAIL), F32),
            pltpu.VMEM((N,), F32),
            pltpu.VMEM((N,), F32),
            pltpu.VMEM((N,), F32),
            pltpu.SemaphoreType.DMA,
            pltpu.SemaphoreType.DMA,
            pltpu.SemaphoreType.DMA,
            pltpu.SemaphoreType.DMA,
            pltpu.SemaphoreType.DMA,
            pltpu.SemaphoreType.DMA,
        ],
    )(_sc_gather_body)
    scatter_m = functools.partial(
        pl.kernel,
        compiler_params=cparams,
        out_type=jax.ShapeDtypeStruct((NC, NP, HID), F32),
        mesh=mesh,
        scratch_types=[
            pltpu.VMEM((NCH, KCH), jnp.int32),
            pltpu.VMEM((KCH, HID), F32),
            pltpu.VMEM((KCH, HID), F32),
            pltpu.VMEM_SHARED((NP, HID), F32),
            pltpu.SemaphoreType.DMA,
            pltpu.SemaphoreType.DMA,
            pltpu.SemaphoreType.DMA,
            pltpu.SemaphoreType.DMA,
        ],
    )(_sc_scatter_m_body)
    scatter_t = functools.partial(
        pl.kernel,
        compiler_params=cparams,
        out_type=jax.ShapeDtypeStruct((NW, TR, HID), F32),
        mesh=mesh,
        scratch_types=[
            pltpu.VMEM((NCH, KCH), jnp.int32),
            pltpu.VMEM((KCH, TAIL), F32),
            pltpu.VMEM((KCH, TAIL), F32),
            pltpu.VMEM((KCH, TAIL), F32),
            pltpu.VMEM((KCH, TAIL), F32),
            pltpu.VMEM((TR, HID), F32),
            pltpu.SemaphoreType.DMA,
            pltpu.SemaphoreType.DMA,
        ],
    )(_sc_scatter_t_body)
    return gather, scatter_m, scatter_t


def _sc_gather(ta, tb, row3, col3, cx, cy, cz):
    return _sc_kernels()[0](ta, tb, row3, col3, cx, cy, cz)


def _sc_scatter_m(m, row3, zrows):
    return _sc_kernels()[1](m, row3, zrows)


def _sc_scatter_t(s16, es, row3):
    return _sc_kernels()[2](s16, es, row3)


BN = 2000
BE = 2000


def _full(shape):
    return pl.BlockSpec(shape, lambda i: (0,) * len(shape))


def _blk(shape, pos=0):
    def imap(i):
        out = [0] * len(shape)
        out[pos] = i
        return tuple(out)
    return pl.BlockSpec(shape, imap)


def _tc_node0(h, cp, wemb, bemb, w1a, be1, w1b):
    return pl.pallas_call(
        _node0_body,
        grid=(N // BN,),
        in_specs=[_blk((BN, HID)), _blk((BN, TAIL)), _full((HID, HID)),
                  _full((1, HID)), _full((HID, HID)), _full((1, HID)),
                  _full((HID, HID))],
        out_specs=[_blk((BN, HID)), _blk((BN, HID)), _blk((BN, HID))],
        out_shape=[jax.ShapeDtypeStruct((N, HID), F32),
                   jax.ShapeDtypeStruct((N, HID), F32),
                   jax.ShapeDtypeStruct((N, HID), F32)],
    )(h, cp, wemb, bemb, w1a, be1, w1b)


def _tc_node(h, cp, vp, p0, p1, pt, wv1, bv1, wv2, bv2, wn1a, wn1b, bn1,
             wn2, bn2, w1a, be1, w1b):
    return pl.pallas_call(
        _node_body,
        grid=(N // BN,),
        in_specs=[_blk((BN, HID)), _blk((BN, TAIL)), _blk((BN, TAIL)),
                  _blk((BN, HID)), _blk((BN, HID)), _blk((BN, NT)),
                  _full((HID, HID)), _full((1, HID)), _full((HID, 1)),
                  _full((1, 1)),
                  _full((HID, HID)), _full((HID, HID)), _full((1, HID)),
                  _full((HID, HID)), _full((1, HID)),
                  _full((HID, HID)), _full((1, HID)), _full((HID, HID))],
        out_specs=[_blk((BN, HID)), _blk((BN, TAIL)), _blk((BN, HID)),
                   _blk((BN, HID))],
        out_shape=[jax.ShapeDtypeStruct((N, HID), F32),
                   jax.ShapeDtypeStruct((N, TAIL), F32),
                   jax.ShapeDtypeStruct((N, HID), F32),
                   jax.ShapeDtypeStruct((N, HID), F32)],
    )(h, cp, vp, p0, p1, pt, wv1, bv1, wv2, bv2, wn1a, wn1b, bn1, wn2, bn2,
      w1a, be1, w1b)


def _tc_node_last(h, cp, vp, pt, wv1, bv1, wv2, bv2):
    return pl.pallas_call(
        _node_last_body,
        grid=(N // BN,),
        in_specs=[_blk((BN, HID)), _blk((BN, TAIL)), _blk((BN, TAIL)),
                  _blk((BN, NT)),
                  _full((HID, HID)), _full((1, HID)), _full((HID, 1)),
                  _full((1, 1))],
        out_specs=_blk((BN, TAIL)),
        out_shape=jax.ShapeDtypeStruct((N, TAIL), F32),
    )(h, cp, vp, pt, wv1, bv1, wv2, bv2)


def _tc_reduce(pt):
    return pl.pallas_call(
        _reduce_body,
        grid=(1,),
        in_specs=[_full((NW, TR, HID))],
        out_specs=_full((TR, HID)),
        out_shape=jax.ShapeDtypeStruct((TR, HID), F32),
    )(pt)


def _tc_edge(ga, gb, es, ea, wr, w1e, we2, be2, wc1, bc1, wc2):
    return pl.pallas_call(
        _edge_body,
        grid=(E // BE,),
        in_specs=[_blk((BE, HID)), _blk((BE, HID)), _blk((BE, TAIL)),
                  _blk((BE, 16)),
                  _full((1, HID)), _full((16, HID)), _full((HID, HID)),
                  _full((1, HID)), _full((HID, HID)), _full((1, HID)),
                  _full((HID, 1))],
        out_specs=[_blk((BE, HID)), _blk((BE, TAIL))],
        out_shape=[jax.ShapeDtypeStruct((E, HID), F32),
                   jax.ShapeDtypeStruct((E, TAIL), F32)],
    )(ga, gb, es, ea, wr, w1e, we2, be2, wc1, bc1, wc2)


def kernel(h, x, edges, vel, edge_attr, params):
    row = edges[0]
    col = edges[1]
    row3 = row.reshape(NW, NCH, KCH)
    col3 = col.reshape(NW, NCH, KCH)
    cp = jnp.pad(x, ((0, 0), (0, TAIL - 3)))
    vp = jnp.pad(vel, ((0, 0), (0, TAIL - 3)))
    zrows = jnp.zeros((SPT, HID), F32)
    r2 = lambda b: b.reshape(1, -1)

    lp = params["layers"]
    p0w = lp[0]
    hh, ta, tb = _tc_node0(
        h, cp, params["emb"]["W"], r2(params["emb"]["b"]),
        p0w["We1"][:HID], r2(p0w["be1"]), p0w["We1"][HID:2 * HID])

    for li in range(4):
        p = lp[li]
        cx, cy, cz = cp[:, 0], cp[:, 1], cp[:, 2]
        ga, gb, es = _sc_gather(ta, tb, row3, col3, cx, cy, cz)
        m, s16 = _tc_edge(ga, gb, es, edge_attr, r2(p["We1"][2 * HID]),
                          p["We1"][2 * HID + 1:], p["We2"], r2(p["be2"]),
                          p["Wc1"], r2(p["bc1"]), p["Wc2"])
        pm = _sc_scatter_m(m, row3, zrows)
        pt = _sc_scatter_t(s16, es, row3)
        ptr = _tc_reduce(pt).reshape(TR * HID)[:N * NT].reshape(N, NT)
        if li < 3:
            nx = lp[li + 1]
            hh, cp, ta, tb = _tc_node(
                hh, cp, vp, pm[0, :N], pm[1, :N], ptr,
                p["Wv1"], r2(p["bv1"]), p["Wv2"], r2(p["bv2"]),
                p["Wn1"][:HID], p["Wn1"][HID:], r2(p["bn1"]),
                p["Wn2"], r2(p["bn2"]),
                nx["We1"][:HID], r2(nx["be1"]), nx["We1"][HID:2 * HID])
        else:
            cp = _tc_node_last(hh, cp, vp, ptr,
                               p["Wv1"], r2(p["bv1"]), p["Wv2"],
                               r2(p["bv2"]))
    return cp[:, :3]


# speedup vs baseline: 4.7284x; 1.1090x over previous
import functools

import jax
import jax.numpy as jnp
from jax import lax
from jax.experimental import pallas as pl
from jax.experimental.pallas import tpu as pltpu
from jax.experimental.pallas import tpu_sc as plsc

N = 10000
E = 320000
HID = 128
TAIL = 16

NC, NS = 2, 16
NW = NC * NS
EPW = E // NW
KCH = 80
NCH = EPW // KCH
NP = 10240
SPT = NP // NS
NT = 4
TR = 320

F32 = jnp.float32
BF16 = jnp.bfloat16


def _silu(x):
    return x * (1.0 / (1.0 + jnp.exp(-x)))


def _node0_body(h, cp, wemb, bemb, w1a, be1, w1b, hh_o, ta_o, tb_o):
    hh = jnp.dot(h[...], wemb[...], preferred_element_type=F32) + bemb[...]
    hh_o[...] = hh
    ta_o[...] = jnp.dot(hh, w1a[...], preferred_element_type=F32) + be1[...]
    tb_o[...] = jnp.dot(hh, w1b[...], preferred_element_type=F32)


def _node_body(h, cp, vp, p0, p1, pt, wv1, bv1, wv2, bv2, wn1a, wn1b, bn1,
               wn2, bn2, w1a, be1, w1b, hn_o, cn_o, ta_o, tb_o):
    hv = h[...]
    aggm = p0[...] + p1[...]
    tl = pt[...]
    cnt = jnp.maximum(tl[:, 3:4], 1.0)
    lane = lax.broadcasted_iota(jnp.int32, (1, TAIL), 1)
    mask3 = (lane < 3).astype(F32)
    tl16 = jnp.concatenate(
        [tl, jnp.zeros((tl.shape[0], TAIL - NT), F32)], axis=1)
    sv = (jnp.dot(_silu(jnp.dot(hv, wv1[...], preferred_element_type=F32)
                        + bv1[...]), wv2[...], preferred_element_type=F32)
          + bv2[...])
    cn = cp[...] + (tl16 * mask3) / cnt + sv * vp[...]
    cn_o[...] = cn
    t = _silu(jnp.dot(hv, wn1a[...], preferred_element_type=F32)
              + jnp.dot(aggm, wn1b[...], preferred_element_type=F32)
              + bn1[...])
    hn = hv + jnp.dot(t, wn2[...], preferred_element_type=F32) + bn2[...]
    hn_o[...] = hn
    ta_o[...] = jnp.dot(hn, w1a[...], preferred_element_type=F32) + be1[...]
    tb_o[...] = jnp.dot(hn, w1b[...], preferred_element_type=F32)


def _node_last_body(h, cp, vp, pt, wv1, bv1, wv2, bv2, cn_o):
    hv = h[...]
    tl = pt[...]
    cnt = jnp.maximum(tl[:, 3:4], 1.0)
    lane = lax.broadcasted_iota(jnp.int32, (1, TAIL), 1)
    mask3 = (lane < 3).astype(F32)
    tl16 = jnp.concatenate(
        [tl, jnp.zeros((tl.shape[0], TAIL - NT), F32)], axis=1)
    sv = (jnp.dot(_silu(jnp.dot(hv, wv1[...], preferred_element_type=F32)
                        + bv1[...]), wv2[...], preferred_element_type=F32)
          + bv2[...])
    cn_o[...] = cp[...] + (tl16 * mask3) / cnt + sv * vp[...]


def _edge_body(g, es, ea, wr, w1e, we2, be2, wc1, bc1, wc2, m_o, s_o):
    radial = es[:, 0:1]
    e1 = (g[...] + radial * wr[...]
          + jnp.dot(ea[...], w1e[...], preferred_element_type=F32))
    m = _silu(jnp.dot(_silu(e1).astype(BF16), we2[...].astype(BF16),
                      preferred_element_type=F32) + be2[...])
    m_o[...] = m
    cm = _silu(jnp.dot(m.astype(BF16), wc1[...].astype(BF16),
                       preferred_element_type=F32) + bc1[...])
    s = jnp.dot(cm, wc2[...], preferred_element_type=F32)
    s_o[...] = jnp.concatenate(
        [s, jnp.zeros((s.shape[0], TAIL - 1), F32)], axis=1)


def _reduce_body(pin, out):
    out[...] = jnp.sum(pin[...], axis=0)


def _sc_gather_body(ta, tb, row, col, cx, cy, cz, g_o, es_o,
                    rowi, coli, bufa0, bufb0, bufa1, bufb1, gsum, esb,
                    cxv, cyv, czv, sga0, sgb0, sga1, sgb1, semw0, semw1):
    wid = lax.axis_index("s") * NC + lax.axis_index("c")
    base = wid * EPW
    pltpu.sync_copy(row.at[pl.ds(base, EPW)], rowi)
    pltpu.sync_copy(col.at[pl.ds(base, EPW)], coli)
    pltpu.sync_copy(cx, cxv)
    pltpu.sync_copy(cy, cyv)
    pltpu.sync_copy(cz, czv)
    iota = lax.iota(jnp.int32, 16)

    def issue_g(c, bufa, bufb, sa, sb):
        pltpu.async_copy(ta.at[rowi.at[pl.ds(c * KCH, KCH)]], bufa, sa)
        pltpu.async_copy(tb.at[coli.at[pl.ds(c * KCH, KCH)]], bufb, sb)

    def wait_g(c, bufa, bufb, sa, sb):
        pltpu.make_async_copy(ta.at[rowi.at[pl.ds(c * KCH, KCH)]], bufa,
                              sa).wait()
        pltpu.make_async_copy(tb.at[coli.at[pl.ds(c * KCH, KCH)]], bufb,
                              sb).wait()

    def compute_es(c, esb):
        for j in range(KCH // 16):
            rv = rowi[pl.ds(c * KCH + j * 16, 16)]
            cv = coli[pl.ds(c * KCH + j * 16, 16)]
            dx = plsc.load_gather(cxv, [rv]) - plsc.load_gather(cxv, [cv])
            dy = plsc.load_gather(cyv, [rv]) - plsc.load_gather(cyv, [cv])
            dz = plsc.load_gather(czv, [rv]) - plsc.load_gather(czv, [cv])
            r = dx * dx + dy * dy + dz * dz
            ri = iota + (j * 16)
            plsc.store_scatter(esb, [ri, iota * 0], r)
            plsc.store_scatter(esb, [ri, iota * 0 + 1], dx)
            plsc.store_scatter(esb, [ri, iota * 0 + 2], dy)
            plsc.store_scatter(esb, [ri, iota * 0 + 3], dz)

    def add_rows(bufa, bufb):
        def radd(r, _):
            for j in range(HID // 16):
                gsum[r, pl.ds(j * 16, 16)] = (bufa[r, pl.ds(j * 16, 16)]
                                              + bufb[r, pl.ds(j * 16, 16)])
            return 0

        lax.fori_loop(0, KCH, radd, 0)

    def drain(c, semw):
        off = base + c * KCH
        wg = pltpu.async_copy(gsum, g_o.at[pl.ds(off, KCH)], semw)
        we = pltpu.async_copy(esb, es_o.at[pl.ds(off, KCH)], semw)
        wg.wait()
        we.wait()

    issue_g(0, bufa0, bufb0, sga0, sgb0)
    issue_g(1, bufa1, bufb1, sga1, sgb1)

    def body(i, _):
        c0 = 2 * i
        c1 = c0 + 1
        compute_es(c0, esb)
        wait_g(c0, bufa0, bufb0, sga0, sgb0)
        add_rows(bufa0, bufb0)
        drain(c0, semw0)
        issue_g(c0 + 2, bufa0, bufb0, sga0, sgb0)
        compute_es(c1, esb)
        wait_g(c1, bufa1, bufb1, sga1, sgb1)
        add_rows(bufa1, bufb1)
        drain(c1, semw1)

        @pl.when(i < (NCH - 1) // 2 - 1)
        def _():
            issue_g(c1 + 2, bufa1, bufb1, sga1, sgb1)

        return 0

    lax.fori_loop(0, (NCH - 1) // 2, body, 0)
    cl = NCH - 1
    compute_es(cl, esb)
    wait_g(cl, bufa0, bufb0, sga0, sgb0)
    add_rows(bufa0, bufb0)
    drain(cl, semw0)


def _sc_scatter_m_body(m, row3, zrows, pm_o, idr, mb0, mb1, accm,
                       sl0, sl1, sa0, sa1):
    cid = lax.axis_index("c")
    sid = lax.axis_index("s")
    wid = sid * NC + cid
    base = wid * EPW

    def issue_l(c, mb, sl):
        pltpu.async_copy(m.at[pl.ds(base + c * KCH, KCH)], mb, sl)

    def wait_l(c, mb, sl):
        pltpu.make_async_copy(m.at[pl.ds(base + c * KCH, KCH)], mb,
                              sl).wait()

    issue_l(0, mb0, sl0)
    issue_l(1, mb1, sl1)
    pltpu.sync_copy(row3.at[wid], idr)
    pltpu.sync_copy(zrows, accm.at[pl.ds(sid * SPT, SPT)])
    plsc.subcore_barrier()

    def sadd(c, mb, sa):
        pltpu.async_copy(mb, accm.at[idr.at[c]], sa, add=True).wait()

    def body(i, _):
        c0 = 2 * i
        c1 = c0 + 1
        wait_l(c0, mb0, sl0)
        sadd(c0, mb0, sa0)
        issue_l(c0 + 2, mb0, sl0)
        wait_l(c1, mb1, sl1)
        sadd(c1, mb1, sa1)

        @pl.when(i < (NCH - 1) // 2 - 1)
        def _():
            issue_l(c1 + 2, mb1, sl1)

        return 0

    lax.fori_loop(0, (NCH - 1) // 2, body, 0)
    cl = NCH - 1
    wait_l(cl, mb0, sl0)
    sadd(cl, mb0, sa0)
    plsc.subcore_barrier()
    pltpu.sync_copy(accm.at[pl.ds(sid * SPT, SPT)],
                    pm_o.at[cid, pl.ds(sid * SPT, SPT)])


def _sc_scatter_t_body(s16, es, row3, pt_o, idr, sb0, eb0, sb1, eb1, acct,
                       ss0, ss1):
    cid = lax.axis_index("c")
    sid = lax.axis_index("s")
    wid = sid * NC + cid
    base = wid * EPW

    def issue_l(c, sb, eb, ss):
        off = base + c * KCH
        pltpu.async_copy(s16.at[pl.ds(off, KCH)], sb, ss)
        pltpu.async_copy(es.at[pl.ds(off, KCH)], eb, ss)

    def wait_l(c, sb, eb, ss):
        off = base + c * KCH
        pltpu.make_async_copy(s16.at[pl.ds(off, KCH)], sb, ss).wait()
        pltpu.make_async_copy(es.at[pl.ds(off, KCH)], eb, ss).wait()

    issue_l(0, sb0, eb0, ss0)
    issue_l(1, sb1, eb1, ss1)
    pltpu.sync_copy(row3.at[wid], idr)
    zero16 = jnp.zeros((16,), F32)

    def zbody(i, _):
        for j in range(HID // 16):
            acct[i, pl.ds(j * 16, 16)] = zero16
        return 0

    lax.fori_loop(0, TR, zbody, 0)
    iota = lax.iota(jnp.int32, 16)
    one16 = zero16 + 1.0

    def process(c, sb, eb):
        for j in range(KCH // 16):
            rv = idr[c, pl.ds(j * 16, 16)]
            ri = iota + (j * 16)
            sv = plsc.load_gather(sb, [ri, iota * 0])
            dx = plsc.load_gather(eb, [ri, iota * 0 + 1])
            dy = plsc.load_gather(eb, [ri, iota * 0 + 2])
            dz = plsc.load_gather(eb, [ri, iota * 0 + 3])
            fb = rv * NT
            for k, v in ((0, dx * sv), (1, dy * sv), (2, dz * sv),
                         (3, one16)):
                fk = fb + k
                plsc.addupdate_scatter(
                    acct, [lax.shift_right_logical(fk, 7), fk & 127], v)

    def body(i, _):
        c0 = 2 * i
        c1 = c0 + 1
        wait_l(c0, sb0, eb0, ss0)
        process(c0, sb0, eb0)
        issue_l(c0 + 2, sb0, eb0, ss0)
        wait_l(c1, sb1, eb1, ss1)
        process(c1, sb1, eb1)

        @pl.when(i < (NCH - 1) // 2 - 1)
        def _():
            issue_l(c1 + 2, sb1, eb1, ss1)

        return 0

    lax.fori_loop(0, (NCH - 1) // 2, body, 0)
    cl = NCH - 1
    wait_l(cl, sb0, eb0, ss0)
    process(cl, sb0, eb0)
    pltpu.sync_copy(acct, pt_o.at[wid])


@functools.cache
def _sc_kernels():
    mesh = plsc.VectorSubcoreMesh(core_axis_name="c", subcore_axis_name="s",
                                  num_cores=NC, num_subcores=NS)
    cparams = pltpu.CompilerParams(needs_layout_passes=False)
    gather = functools.partial(
        pl.kernel,
        compiler_params=cparams,
        out_type=(jax.ShapeDtypeStruct((E, HID), F32),
                  jax.ShapeDtypeStruct((E, TAIL), F32)),
        mesh=mesh,
        scratch_types=[
            pltpu.VMEM((EPW,), jnp.int32),
            pltpu.VMEM((EPW,), jnp.int32),
            pltpu.VMEM((KCH, HID), F32),
            pltpu.VMEM((KCH, HID), F32),
            pltpu.VMEM((KCH, HID), F32),
            pltpu.VMEM((KCH, HID), F32),
            pltpu.VMEM((KCH, HID), F32),
            pltpu.VMEM((KCH, TAIL), F32),
            pltpu.VMEM((N,), F32),
            pltpu.VMEM((N,), F32),
            pltpu.VMEM((N,), F32),
            pltpu.SemaphoreType.DMA,
            pltpu.SemaphoreType.DMA,
            pltpu.SemaphoreType.DMA,
            pltpu.SemaphoreType.DMA,
            pltpu.SemaphoreType.DMA,
            pltpu.SemaphoreType.DMA,
        ],
    )(_sc_gather_body)
    scatter_m = functools.partial(
        pl.kernel,
        compiler_params=cparams,
        out_type=jax.ShapeDtypeStruct((NC, NP, HID), F32),
        mesh=mesh,
        scratch_types=[
            pltpu.VMEM((NCH, KCH), jnp.int32),
            pltpu.VMEM((KCH, HID), F32),
            pltpu.VMEM((KCH, HID), F32),
            pltpu.VMEM_SHARED((NP, HID), F32),
            pltpu.SemaphoreType.DMA,
            pltpu.SemaphoreType.DMA,
            pltpu.SemaphoreType.DMA,
            pltpu.SemaphoreType.DMA,
        ],
    )(_sc_scatter_m_body)
    scatter_t = functools.partial(
        pl.kernel,
        compiler_params=cparams,
        out_type=jax.ShapeDtypeStruct((NW, TR, HID), F32),
        mesh=mesh,
        scratch_types=[
            pltpu.VMEM((NCH, KCH), jnp.int32),
            pltpu.VMEM((KCH, TAIL), F32),
            pltpu.VMEM((KCH, TAIL), F32),
            pltpu.VMEM((KCH, TAIL), F32),
            pltpu.VMEM((KCH, TAIL), F32),
            pltpu.VMEM((TR, HID), F32),
            pltpu.SemaphoreType.DMA,
            pltpu.SemaphoreType.DMA,
        ],
    )(_sc_scatter_t_body)
    return gather, scatter_m, scatter_t


def _sc_gather(ta, tb, row, col, cx, cy, cz):
    return _sc_kernels()[0](ta, tb, row, col, cx, cy, cz)


def _sc_scatter_m(m, row3, zrows):
    return _sc_kernels()[1](m, row3, zrows)


def _sc_scatter_t(s16, es, row3):
    return _sc_kernels()[2](s16, es, row3)


BN = 2000
BE = 2000


def _full(shape):
    return pl.BlockSpec(shape, lambda i: (0,) * len(shape))


def _blk(shape, pos=0):
    def imap(i):
        out = [0] * len(shape)
        out[pos] = i
        return tuple(out)
    return pl.BlockSpec(shape, imap)


def _tc_node0(h, cp, wemb, bemb, w1a, be1, w1b):
    return pl.pallas_call(
        _node0_body,
        grid=(N // BN,),
        in_specs=[_blk((BN, HID)), _blk((BN, TAIL)), _full((HID, HID)),
                  _full((1, HID)), _full((HID, HID)), _full((1, HID)),
                  _full((HID, HID))],
        out_specs=[_blk((BN, HID)), _blk((BN, HID)), _blk((BN, HID))],
        out_shape=[jax.ShapeDtypeStruct((N, HID), F32),
                   jax.ShapeDtypeStruct((N, HID), F32),
                   jax.ShapeDtypeStruct((N, HID), F32)],
    )(h, cp, wemb, bemb, w1a, be1, w1b)


def _tc_node(h, cp, vp, p0, p1, pt, wv1, bv1, wv2, bv2, wn1a, wn1b, bn1,
             wn2, bn2, w1a, be1, w1b):
    return pl.pallas_call(
        _node_body,
        grid=(N // BN,),
        in_specs=[_blk((BN, HID)), _blk((BN, TAIL)), _blk((BN, TAIL)),
                  _blk((BN, HID)), _blk((BN, HID)), _blk((BN, NT)),
                  _full((HID, HID)), _full((1, HID)), _full((HID, 1)),
                  _full((1, 1)),
                  _full((HID, HID)), _full((HID, HID)), _full((1, HID)),
                  _full((HID, HID)), _full((1, HID)),
                  _full((HID, HID)), _full((1, HID)), _full((HID, HID))],
        out_specs=[_blk((BN, HID)), _blk((BN, TAIL)), _blk((BN, HID)),
                   _blk((BN, HID))],
        out_shape=[jax.ShapeDtypeStruct((N, HID), F32),
                   jax.ShapeDtypeStruct((N, TAIL), F32),
                   jax.ShapeDtypeStruct((N, HID), F32),
                   jax.ShapeDtypeStruct((N, HID), F32)],
    )(h, cp, vp, p0, p1, pt, wv1, bv1, wv2, bv2, wn1a, wn1b, bn1, wn2, bn2,
      w1a, be1, w1b)


def _tc_node_last(h, cp, vp, pt, wv1, bv1, wv2, bv2):
    return pl.pallas_call(
        _node_last_body,
        grid=(N // BN,),
        in_specs=[_blk((BN, HID)), _blk((BN, TAIL)), _blk((BN, TAIL)),
                  _blk((BN, NT)),
                  _full((HID, HID)), _full((1, HID)), _full((HID, 1)),
                  _full((1, 1))],
        out_specs=_blk((BN, TAIL)),
        out_shape=jax.ShapeDtypeStruct((N, TAIL), F32),
    )(h, cp, vp, pt, wv1, bv1, wv2, bv2)


def _tc_reduce(pt):
    return pl.pallas_call(
        _reduce_body,
        grid=(1,),
        in_specs=[_full((NW, TR, HID))],
        out_specs=_full((TR, HID)),
        out_shape=jax.ShapeDtypeStruct((TR, HID), F32),
    )(pt)


def _tc_edge(g, es, ea, wr, w1e, we2, be2, wc1, bc1, wc2):
    return pl.pallas_call(
        _edge_body,
        grid=(E // BE,),
        in_specs=[_blk((BE, HID)), _blk((BE, TAIL)),
                  _blk((BE, 16)),
                  _full((1, HID)), _full((16, HID)), _full((HID, HID)),
                  _full((1, HID)), _full((HID, HID)), _full((1, HID)),
                  _full((HID, 1))],
        out_specs=[_blk((BE, HID)), _blk((BE, TAIL))],
        out_shape=[jax.ShapeDtypeStruct((E, HID), F32),
                   jax.ShapeDtypeStruct((E, TAIL), F32)],
    )(g, es, ea, wr, w1e, we2, be2, wc1, bc1, wc2)


def kernel(h, x, edges, vel, edge_attr, params):
    row = edges[0]
    col = edges[1]
    row3 = row.reshape(NW, NCH, KCH)
    col3 = col.reshape(NW, NCH, KCH)
    cp = jnp.pad(x, ((0, 0), (0, TAIL - 3)))
    vp = jnp.pad(vel, ((0, 0), (0, TAIL - 3)))
    zrows = jnp.zeros((SPT, HID), F32)
    r2 = lambda b: b.reshape(1, -1)

    lp = params["layers"]
    p0w = lp[0]
    hh, ta, tb = _tc_node0(
        h, cp, params["emb"]["W"], r2(params["emb"]["b"]),
        p0w["We1"][:HID], r2(p0w["be1"]), p0w["We1"][HID:2 * HID])

    for li in range(4):
        p = lp[li]
        cx, cy, cz = cp[:, 0], cp[:, 1], cp[:, 2]
        g, es = _sc_gather(ta, tb, row, col, cx, cy, cz)
        m, s16 = _tc_edge(g, es, edge_attr, r2(p["We1"][2 * HID]),
                          p["We1"][2 * HID + 1:], p["We2"], r2(p["be2"]),
                          p["Wc1"], r2(p["bc1"]), p["Wc2"])
        pm = _sc_scatter_m(m, row3, zrows)
        pt = _sc_scatter_t(s16, es, row3)
        ptr = _tc_reduce(pt).reshape(TR * HID)[:N * NT].reshape(N, NT)
        if li < 3:
            nx = lp[li + 1]
            hh, cp, ta, tb = _tc_node(
                hh, cp, vp, pm[0, :N], pm[1, :N], ptr,
                p["Wv1"], r2(p["bv1"]), p["Wv2"], r2(p["bv2"]),
                p["Wn1"][:HID], p["Wn1"][HID:], r2(p["bn1"]),
                p["Wn2"], r2(p["bn2"]),
                nx["We1"][:HID], r2(nx["be1"]), nx["We1"][HID:2 * HID])
        else:
            cp = _tc_node_last(hh, cp, vp, ptr,
                               p["Wv1"], r2(p["bv1"]), p["Wv2"],
                               r2(p["bv2"]))
    return cp[:, :3]


# speedup vs baseline: 5.0517x; 1.0684x over previous
import functools

import jax
import jax.numpy as jnp
from jax import lax
from jax.experimental import pallas as pl
from jax.experimental.pallas import tpu as pltpu
from jax.experimental.pallas import tpu_sc as plsc

N = 10000
E = 320000
HID = 128
TAIL = 16

NC, NS = 2, 16
NW = NC * NS
EPW = E // NW
KCH = 80
NCH = EPW // KCH
NP = 10240
SPT = NP // NS
NT = 4
TR = 320

F32 = jnp.float32
BF16 = jnp.bfloat16


def _silu(x):
    return x * (1.0 / (1.0 + jnp.exp(-x)))


def _node0_body(h, cp, wemb, bemb, w1a, be1, w1b, hh_o, ta_o, tb_o):
    hh = jnp.dot(h[...], wemb[...], preferred_element_type=F32) + bemb[...]
    hh_o[...] = hh
    ta_o[...] = jnp.dot(hh, w1a[...], preferred_element_type=F32) + be1[...]
    tb_o[...] = jnp.dot(hh, w1b[...], preferred_element_type=F32)


def _node_body(h, cp, vp, p0, p1, pt, wv1, bv1, wv2, bv2, wn1a, wn1b, bn1,
               wn2, bn2, w1a, be1, w1b, hn_o, cn_o, ta_o, tb_o):
    hv = h[...]
    aggm = p0[...] + p1[...]
    tl = pt[...]
    cnt = jnp.maximum(tl[:, 3:4], 1.0)
    lane = lax.broadcasted_iota(jnp.int32, (1, TAIL), 1)
    mask3 = (lane < 3).astype(F32)
    tl16 = jnp.concatenate(
        [tl, jnp.zeros((tl.shape[0], TAIL - NT), F32)], axis=1)
    sv = (jnp.dot(_silu(jnp.dot(hv, wv1[...], preferred_element_type=F32)
                        + bv1[...]), wv2[...], preferred_element_type=F32)
          + bv2[...])
    cn = cp[...] + (tl16 * mask3) / cnt + sv * vp[...]
    cn_o[...] = cn
    t = _silu(jnp.dot(hv, wn1a[...], preferred_element_type=F32)
              + jnp.dot(aggm, wn1b[...], preferred_element_type=F32)
              + bn1[...])
    hn = hv + jnp.dot(t, wn2[...], preferred_element_type=F32) + bn2[...]
    hn_o[...] = hn
    ta_o[...] = jnp.dot(hn, w1a[...], preferred_element_type=F32) + be1[...]
    tb_o[...] = jnp.dot(hn, w1b[...], preferred_element_type=F32)


def _node_last_body(h, cp, vp, pt, wv1, bv1, wv2, bv2, cn_o):
    hv = h[...]
    tl = pt[...]
    cnt = jnp.maximum(tl[:, 3:4], 1.0)
    lane = lax.broadcasted_iota(jnp.int32, (1, TAIL), 1)
    mask3 = (lane < 3).astype(F32)
    tl16 = jnp.concatenate(
        [tl, jnp.zeros((tl.shape[0], TAIL - NT), F32)], axis=1)
    sv = (jnp.dot(_silu(jnp.dot(hv, wv1[...], preferred_element_type=F32)
                        + bv1[...]), wv2[...], preferred_element_type=F32)
          + bv2[...])
    cn_o[...] = cp[...] + (tl16 * mask3) / cnt + sv * vp[...]


def _edge_body(g, es, ea, wr, w1e, we2, be2, wc1, bc1, wc2, m_o, s_o):
    radial = es[:, 0:1]
    e1 = (g[...] + radial * wr[...]
          + jnp.dot(ea[...], w1e[...], preferred_element_type=F32))
    m = _silu(jnp.dot(_silu(e1).astype(BF16), we2[...].astype(BF16),
                      preferred_element_type=F32) + be2[...])
    m_o[...] = m
    cm = _silu(jnp.dot(m.astype(BF16), wc1[...].astype(BF16),
                       preferred_element_type=F32) + bc1[...])
    s = jnp.dot(cm, wc2[...], preferred_element_type=F32)
    s_o[...] = jnp.concatenate(
        [s, es[:, 1:4], jnp.zeros((s.shape[0], TAIL - 4), F32)], axis=1)


def _reduce_body(pin, out):
    out[...] = jnp.sum(pin[...], axis=0)


def _sc_gather_body(ta, tb, row, col, cx, cy, cz, g_o, es_o,
                    rowi, coli, bufa0, bufb0, bufa1, bufb1, gsum, esb,
                    cxv, cyv, czv, sga0, sgb0, sga1, sgb1, semw0, semw1):
    wid = lax.axis_index("s") * NC + lax.axis_index("c")
    base = wid * EPW
    pltpu.sync_copy(row.at[pl.ds(base, EPW)], rowi)
    pltpu.sync_copy(col.at[pl.ds(base, EPW)], coli)
    pltpu.sync_copy(cx, cxv)
    pltpu.sync_copy(cy, cyv)
    pltpu.sync_copy(cz, czv)
    iota = lax.iota(jnp.int32, 16)

    def issue_g(c, bufa, bufb, sa, sb):
        pltpu.async_copy(ta.at[rowi.at[pl.ds(c * KCH, KCH)]], bufa, sa)
        pltpu.async_copy(tb.at[coli.at[pl.ds(c * KCH, KCH)]], bufb, sb)

    def wait_g(c, bufa, bufb, sa, sb):
        pltpu.make_async_copy(ta.at[rowi.at[pl.ds(c * KCH, KCH)]], bufa,
                              sa).wait()
        pltpu.make_async_copy(tb.at[coli.at[pl.ds(c * KCH, KCH)]], bufb,
                              sb).wait()

    def compute_es(c, esb):
        for j in range(KCH // 16):
            rv = rowi[pl.ds(c * KCH + j * 16, 16)]
            cv = coli[pl.ds(c * KCH + j * 16, 16)]
            dx = plsc.load_gather(cxv, [rv]) - plsc.load_gather(cxv, [cv])
            dy = plsc.load_gather(cyv, [rv]) - plsc.load_gather(cyv, [cv])
            dz = plsc.load_gather(czv, [rv]) - plsc.load_gather(czv, [cv])
            r = dx * dx + dy * dy + dz * dz
            ri = iota + (j * 16)
            plsc.store_scatter(esb, [ri, iota * 0], r)
            plsc.store_scatter(esb, [ri, iota * 0 + 1], dx)
            plsc.store_scatter(esb, [ri, iota * 0 + 2], dy)
            plsc.store_scatter(esb, [ri, iota * 0 + 3], dz)

    def add_rows(bufa, bufb):
        def radd(r, _):
            for j in range(HID // 16):
                gsum[r, pl.ds(j * 16, 16)] = (bufa[r, pl.ds(j * 16, 16)]
                                              + bufb[r, pl.ds(j * 16, 16)])
            return 0

        lax.fori_loop(0, KCH, radd, 0)

    def drain(c, semw):
        off = base + c * KCH
        wg = pltpu.async_copy(gsum, g_o.at[pl.ds(off, KCH)], semw)
        we = pltpu.async_copy(esb, es_o.at[pl.ds(off, KCH)], semw)
        wg.wait()
        we.wait()

    issue_g(0, bufa0, bufb0, sga0, sgb0)
    issue_g(1, bufa1, bufb1, sga1, sgb1)

    def body(i, _):
        c0 = 2 * i
        c1 = c0 + 1
        compute_es(c0, esb)
        wait_g(c0, bufa0, bufb0, sga0, sgb0)
        add_rows(bufa0, bufb0)
        drain(c0, semw0)
        issue_g(c0 + 2, bufa0, bufb0, sga0, sgb0)
        compute_es(c1, esb)
        wait_g(c1, bufa1, bufb1, sga1, sgb1)
        add_rows(bufa1, bufb1)
        drain(c1, semw1)

        @pl.when(i < (NCH - 1) // 2 - 1)
        def _():
            issue_g(c1 + 2, bufa1, bufb1, sga1, sgb1)

        return 0

    lax.fori_loop(0, (NCH - 1) // 2, body, 0)
    cl = NCH - 1
    compute_es(cl, esb)
    wait_g(cl, bufa0, bufb0, sga0, sgb0)
    add_rows(bufa0, bufb0)
    drain(cl, semw0)


def _sc_scatter_m_body(m, row3, zrows, pm_o, idr, mb0, mb1, accm,
                       sl0, sl1, sa0, sa1):
    cid = lax.axis_index("c")
    sid = lax.axis_index("s")
    wid = sid * NC + cid
    base = wid * EPW

    def issue_l(c, mb, sl):
        pltpu.async_copy(m.at[pl.ds(base + c * KCH, KCH)], mb, sl)

    def wait_l(c, mb, sl):
        pltpu.make_async_copy(m.at[pl.ds(base + c * KCH, KCH)], mb,
                              sl).wait()

    issue_l(0, mb0, sl0)
    issue_l(1, mb1, sl1)
    pltpu.sync_copy(row3.at[wid], idr)
    pltpu.sync_copy(zrows, accm.at[pl.ds(sid * SPT, SPT)])
    plsc.subcore_barrier()

    def sadd(c, mb, sa):
        pltpu.async_copy(mb, accm.at[idr.at[c]], sa, add=True).wait()

    def body(i, _):
        c0 = 2 * i
        c1 = c0 + 1
        wait_l(c0, mb0, sl0)
        sadd(c0, mb0, sa0)
        issue_l(c0 + 2, mb0, sl0)
        wait_l(c1, mb1, sl1)
        sadd(c1, mb1, sa1)

        @pl.when(i < (NCH - 1) // 2 - 1)
        def _():
            issue_l(c1 + 2, mb1, sl1)

        return 0

    lax.fori_loop(0, (NCH - 1) // 2, body, 0)
    cl = NCH - 1
    wait_l(cl, mb0, sl0)
    sadd(cl, mb0, sa0)
    plsc.subcore_barrier()
    pltpu.sync_copy(accm.at[pl.ds(sid * SPT, SPT)],
                    pm_o.at[cid, pl.ds(sid * SPT, SPT)])


def _sc_scatter_t_body(s16, row3, pt_o, idr, sb0, sb1, acct, ss0, ss1):
    cid = lax.axis_index("c")
    sid = lax.axis_index("s")
    wid = sid * NC + cid
    base = wid * EPW

    def issue_l(c, sb, ss):
        pltpu.async_copy(s16.at[pl.ds(base + c * KCH, KCH)], sb, ss)

    def wait_l(c, sb, ss):
        pltpu.make_async_copy(s16.at[pl.ds(base + c * KCH, KCH)], sb,
                              ss).wait()

    issue_l(0, sb0, ss0)
    issue_l(1, sb1, ss1)
    pltpu.sync_copy(row3.at[wid], idr)
    zero16 = jnp.zeros((16,), F32)

    def zbody(i, _):
        for j in range(HID // 16):
            acct[i, pl.ds(j * 16, 16)] = zero16
        return 0

    lax.fori_loop(0, TR, zbody, 0)
    iota = lax.iota(jnp.int32, 16)
    one16 = zero16 + 1.0

    def process(c, sb):
        for j in range(KCH // 16):
            rv = idr[c, pl.ds(j * 16, 16)]
            ri = iota + (j * 16)
            sv = plsc.load_gather(sb, [ri, iota * 0])
            dx = plsc.load_gather(sb, [ri, iota * 0 + 1])
            dy = plsc.load_gather(sb, [ri, iota * 0 + 2])
            dz = plsc.load_gather(sb, [ri, iota * 0 + 3])
            fb = rv * NT
            for k, v in ((0, dx * sv), (1, dy * sv), (2, dz * sv),
                         (3, one16)):
                fk = fb + k
                plsc.addupdate_scatter(
                    acct, [lax.shift_right_logical(fk, 7), fk & 127], v)

    def body(i, _):
        c0 = 2 * i
        c1 = c0 + 1
        wait_l(c0, sb0, ss0)
        process(c0, sb0)
        issue_l(c0 + 2, sb0, ss0)
        wait_l(c1, sb1, ss1)
        process(c1, sb1)

        @pl.when(i < (NCH - 1) // 2 - 1)
        def _():
            issue_l(c1 + 2, sb1, ss1)

        return 0

    lax.fori_loop(0, (NCH - 1) // 2, body, 0)
    cl = NCH - 1
    wait_l(cl, sb0, ss0)
    process(cl, sb0)
    pltpu.sync_copy(acct, pt_o.at[wid])


@functools.cache
def _sc_kernels():
    mesh = plsc.VectorSubcoreMesh(core_axis_name="c", subcore_axis_name="s",
                                  num_cores=NC, num_subcores=NS)
    cparams = pltpu.CompilerParams(needs_layout_passes=False)
    gather = functools.partial(
        pl.kernel,
        compiler_params=cparams,
        out_type=(jax.ShapeDtypeStruct((E, HID), F32),
                  jax.ShapeDtypeStruct((E, TAIL), F32)),
        mesh=mesh,
        scratch_types=[
            pltpu.VMEM((EPW,), jnp.int32),
            pltpu.VMEM((EPW,), jnp.int32),
            pltpu.VMEM((KCH, HID), F32),
            pltpu.VMEM((KCH, HID), F32),
            pltpu.VMEM((KCH, HID), F32),
            pltpu.VMEM((KCH, HID), F32),
            pltpu.VMEM((KCH, HID), F32),
            pltpu.VMEM((KCH, TAIL), F32),
            pltpu.VMEM((N,), F32),
            pltpu.VMEM((N,), F32),
            pltpu.VMEM((N,), F32),
            pltpu.SemaphoreType.DMA,
            pltpu.SemaphoreType.DMA,
            pltpu.SemaphoreType.DMA,
            pltpu.SemaphoreType.DMA,
            pltpu.SemaphoreType.DMA,
            pltpu.SemaphoreType.DMA,
        ],
    )(_sc_gather_body)
    scatter_m = functools.partial(
        pl.kernel,
        compiler_params=cparams,
        out_type=jax.ShapeDtypeStruct((NC, NP, HID), F32),
        mesh=mesh,
        scratch_types=[
            pltpu.VMEM((NCH, KCH), jnp.int32),
            pltpu.VMEM((KCH, HID), F32),
            pltpu.VMEM((KCH, HID), F32),
            pltpu.VMEM_SHARED((NP, HID), F32),
            pltpu.SemaphoreType.DMA,
            pltpu.SemaphoreType.DMA,
            pltpu.SemaphoreType.DMA,
            pltpu.SemaphoreType.DMA,
        ],
    )(_sc_scatter_m_body)
    scatter_t = functools.partial(
        pl.kernel,
        compiler_params=cparams,
        out_type=jax.ShapeDtypeStruct((NW, TR, HID), F32),
        mesh=mesh,
        scratch_types=[
            pltpu.VMEM((NCH, KCH), jnp.int32),
            pltpu.VMEM((KCH, TAIL), F32),
            pltpu.VMEM((KCH, TAIL), F32),
            pltpu.VMEM((TR, HID), F32),
            pltpu.SemaphoreType.DMA,
            pltpu.SemaphoreType.DMA,
        ],
    )(_sc_scatter_t_body)
    return gather, scatter_m, scatter_t


def _sc_gather(ta, tb, row, col, cx, cy, cz):
    return _sc_kernels()[0](ta, tb, row, col, cx, cy, cz)


def _sc_scatter_m(m, row3, zrows):
    return _sc_kernels()[1](m, row3, zrows)


def _sc_scatter_t(s16, row3):
    return _sc_kernels()[2](s16, row3)


BN = 2000
BE = 2000


def _full(shape):
    return pl.BlockSpec(shape, lambda i: (0,) * len(shape))


def _blk(shape, pos=0):
    def imap(i):
        out = [0] * len(shape)
        out[pos] = i
        return tuple(out)
    return pl.BlockSpec(shape, imap)


def _tc_node0(h, cp, wemb, bemb, w1a, be1, w1b):
    return pl.pallas_call(
        _node0_body,
        grid=(N // BN,),
        in_specs=[_blk((BN, HID)), _blk((BN, TAIL)), _full((HID, HID)),
                  _full((1, HID)), _full((HID, HID)), _full((1, HID)),
                  _full((HID, HID))],
        out_specs=[_blk((BN, HID)), _blk((BN, HID)), _blk((BN, HID))],
        out_shape=[jax.ShapeDtypeStruct((N, HID), F32),
                   jax.ShapeDtypeStruct((N, HID), F32),
                   jax.ShapeDtypeStruct((N, HID), F32)],
    )(h, cp, wemb, bemb, w1a, be1, w1b)


def _tc_node(h, cp, vp, p0, p1, pt, wv1, bv1, wv2, bv2, wn1a, wn1b, bn1,
             wn2, bn2, w1a, be1, w1b):
    return pl.pallas_call(
        _node_body,
        grid=(N // BN,),
        in_specs=[_blk((BN, HID)), _blk((BN, TAIL)), _blk((BN, TAIL)),
                  _blk((BN, HID)), _blk((BN, HID)), _blk((BN, NT)),
                  _full((HID, HID)), _full((1, HID)), _full((HID, 1)),
                  _full((1, 1)),
                  _full((HID, HID)), _full((HID, HID)), _full((1, HID)),
                  _full((HID, HID)), _full((1, HID)),
                  _full((HID, HID)), _full((1, HID)), _full((HID, HID))],
        out_specs=[_blk((BN, HID)), _blk((BN, TAIL)), _blk((BN, HID)),
                   _blk((BN, HID))],
        out_shape=[jax.ShapeDtypeStruct((N, HID), F32),
                   jax.ShapeDtypeStruct((N, TAIL), F32),
                   jax.ShapeDtypeStruct((N, HID), F32),
                   jax.ShapeDtypeStruct((N, HID), F32)],
    )(h, cp, vp, p0, p1, pt, wv1, bv1, wv2, bv2, wn1a, wn1b, bn1, wn2, bn2,
      w1a, be1, w1b)


def _tc_node_last(h, cp, vp, pt, wv1, bv1, wv2, bv2):
    return pl.pallas_call(
        _node_last_body,
        grid=(N // BN,),
        in_specs=[_blk((BN, HID)), _blk((BN, TAIL)), _blk((BN, TAIL)),
                  _blk((BN, NT)),
                  _full((HID, HID)), _full((1, HID)), _full((HID, 1)),
                  _full((1, 1))],
        out_specs=_blk((BN, TAIL)),
        out_shape=jax.ShapeDtypeStruct((N, TAIL), F32),
    )(h, cp, vp, pt, wv1, bv1, wv2, bv2)


def _tc_reduce(pt):
    return pl.pallas_call(
        _reduce_body,
        grid=(1,),
        in_specs=[_full((NW, TR, HID))],
        out_specs=_full((TR, HID)),
        out_shape=jax.ShapeDtypeStruct((TR, HID), F32),
    )(pt)


def _tc_edge(g, es, ea, wr, w1e, we2, be2, wc1, bc1, wc2):
    return pl.pallas_call(
        _edge_body,
        grid=(E // BE,),
        in_specs=[_blk((BE, HID)), _blk((BE, TAIL)),
                  _blk((BE, 16)),
                  _full((1, HID)), _full((16, HID)), _full((HID, HID)),
                  _full((1, HID)), _full((HID, HID)), _full((1, HID)),
                  _full((HID, 1))],
        out_specs=[_blk((BE, HID)), _blk((BE, TAIL))],
        out_shape=[jax.ShapeDtypeStruct((E, HID), F32),
                   jax.ShapeDtypeStruct((E, TAIL), F32)],
    )(g, es, ea, wr, w1e, we2, be2, wc1, bc1, wc2)


def kernel(h, x, edges, vel, edge_attr, params):
    row = edges[0]
    col = edges[1]
    row3 = row.reshape(NW, NCH, KCH)
    col3 = col.reshape(NW, NCH, KCH)
    cp = jnp.pad(x, ((0, 0), (0, TAIL - 3)))
    vp = jnp.pad(vel, ((0, 0), (0, TAIL - 3)))
    zrows = jnp.zeros((SPT, HID), F32)
    r2 = lambda b: b.reshape(1, -1)

    lp = params["layers"]
    p0w = lp[0]
    hh, ta, tb = _tc_node0(
        h, cp, params["emb"]["W"], r2(params["emb"]["b"]),
        p0w["We1"][:HID], r2(p0w["be1"]), p0w["We1"][HID:2 * HID])

    for li in range(4):
        p = lp[li]
        cx, cy, cz = cp[:, 0], cp[:, 1], cp[:, 2]
        g, es = _sc_gather(ta, tb, row, col, cx, cy, cz)
        m, s16 = _tc_edge(g, es, edge_attr, r2(p["We1"][2 * HID]),
                          p["We1"][2 * HID + 1:], p["We2"], r2(p["be2"]),
                          p["Wc1"], r2(p["bc1"]), p["Wc2"])
        pm = _sc_scatter_m(m, row3, zrows)
        pt = _sc_scatter_t(s16, row3)
        ptr = _tc_reduce(pt).reshape(TR * HID)[:N * NT].reshape(N, NT)
        if li < 3:
            nx = lp[li + 1]
            hh, cp, ta, tb = _tc_node(
                hh, cp, vp, pm[0, :N], pm[1, :N], ptr,
                p["Wv1"], r2(p["bv1"]), p["Wv2"], r2(p["bv2"]),
                p["Wn1"][:HID], p["Wn1"][HID:], r2(p["bn1"]),
                p["Wn2"], r2(p["bn2"]),
                nx["We1"][:HID], r2(nx["be1"]), nx["We1"][HID:2 * HID])
        else:
            cp = _tc_node_last(hh, cp, vp, ptr,
                               p["Wv1"], r2(p["bv1"]), p["Wv2"],
                               r2(p["bv2"]))
    return cp[:, :3]


# speedup vs baseline: 5.3415x; 1.0574x over previous
import functools

import jax
import jax.numpy as jnp
from jax import lax
from jax.experimental import pallas as pl
from jax.experimental.pallas import tpu as pltpu
from jax.experimental.pallas import tpu_sc as plsc

N = 10000
E = 320000
HID = 128
TAIL = 16

NC, NS = 2, 16
NW = NC * NS
EPW = E // NW
KCH = 80
NCH = EPW // KCH
NP = 10240
SPT = NP // NS
NT = 4
TR = 320

F32 = jnp.float32
BF16 = jnp.bfloat16


def _silu(x):
    return x * (1.0 / (1.0 + jnp.exp(-x)))


def _node0_body(h, cp, wemb, bemb, w1a, be1, w1b, hh_o, ta_o, tb_o):
    hh = jnp.dot(h[...], wemb[...], preferred_element_type=F32) + bemb[...]
    hh_o[...] = hh
    ta_o[...] = jnp.dot(hh, w1a[...], preferred_element_type=F32) + be1[...]
    tb_o[...] = jnp.dot(hh, w1b[...], preferred_element_type=F32)


def _node_body(h, cp, vp, p0, p1, p2, p3, pt, wv1, bv1, wv2, bv2, wn1a,
               wn1b, bn1, wn2, bn2, w1a, be1, w1b, hn_o, cn_o, ta_o, tb_o):
    hv = h[...]
    aggm = (p0[...] + p1[...]) + (p2[...] + p3[...])
    tl = pt[...]
    cnt = jnp.maximum(tl[:, 3:4], 1.0)
    lane = lax.broadcasted_iota(jnp.int32, (1, TAIL), 1)
    mask3 = (lane < 3).astype(F32)
    tl16 = jnp.concatenate(
        [tl, jnp.zeros((tl.shape[0], TAIL - NT), F32)], axis=1)
    sv = (jnp.dot(_silu(jnp.dot(hv, wv1[...], preferred_element_type=F32)
                        + bv1[...]), wv2[...], preferred_element_type=F32)
          + bv2[...])
    cn = cp[...] + (tl16 * mask3) / cnt + sv * vp[...]
    cn_o[...] = cn
    t = _silu(jnp.dot(hv, wn1a[...], preferred_element_type=F32)
              + jnp.dot(aggm, wn1b[...], preferred_element_type=F32)
              + bn1[...])
    hn = hv + jnp.dot(t, wn2[...], preferred_element_type=F32) + bn2[...]
    hn_o[...] = hn
    ta_o[...] = jnp.dot(hn, w1a[...], preferred_element_type=F32) + be1[...]
    tb_o[...] = jnp.dot(hn, w1b[...], preferred_element_type=F32)


def _node_last_body(h, cp, vp, pt, wv1, bv1, wv2, bv2, cn_o):
    hv = h[...]
    tl = pt[...]
    cnt = jnp.maximum(tl[:, 3:4], 1.0)
    lane = lax.broadcasted_iota(jnp.int32, (1, TAIL), 1)
    mask3 = (lane < 3).astype(F32)
    tl16 = jnp.concatenate(
        [tl, jnp.zeros((tl.shape[0], TAIL - NT), F32)], axis=1)
    sv = (jnp.dot(_silu(jnp.dot(hv, wv1[...], preferred_element_type=F32)
                        + bv1[...]), wv2[...], preferred_element_type=F32)
          + bv2[...])
    cn_o[...] = cp[...] + (tl16 * mask3) / cnt + sv * vp[...]


def _edge_body(g, es, ea, wr, w1e, we2, be2, wc1, bc1, wc2, m_o, s_o):
    radial = es[:, 0:1]
    e1 = (g[...] + radial * wr[...]
          + jnp.dot(ea[...], w1e[...], preferred_element_type=F32))
    m = _silu(jnp.dot(_silu(e1).astype(BF16), we2[...].astype(BF16),
                      preferred_element_type=F32) + be2[...])
    m_o[...] = m
    cm = _silu(jnp.dot(m.astype(BF16), wc1[...].astype(BF16),
                       preferred_element_type=F32) + bc1[...])
    s = jnp.dot(cm, wc2[...], preferred_element_type=F32)
    s_o[...] = jnp.concatenate(
        [s, es[:, 1:4], jnp.zeros((s.shape[0], TAIL - 4), F32)], axis=1)


def _edge_body_nom(g, es, ea, wr, w1e, we2, be2, wc1, bc1, wc2, s_o):
    radial = es[:, 0:1]
    e1 = (g[...] + radial * wr[...]
          + jnp.dot(ea[...], w1e[...], preferred_element_type=F32))
    m = _silu(jnp.dot(_silu(e1).astype(BF16), we2[...].astype(BF16),
                      preferred_element_type=F32) + be2[...])
    cm = _silu(jnp.dot(m.astype(BF16), wc1[...].astype(BF16),
                       preferred_element_type=F32) + bc1[...])
    s = jnp.dot(cm, wc2[...], preferred_element_type=F32)
    s_o[...] = jnp.concatenate(
        [s, es[:, 1:4], jnp.zeros((s.shape[0], TAIL - 4), F32)], axis=1)


def _reduce_body(pa, pb, out):
    out[...] = jnp.sum(pa[...], axis=0) + jnp.sum(pb[...], axis=0)


def _make_gather_body(epw, nch):
    pairs = (nch - 1) // 2
    odd = nch % 2 == 1

    def body_fn(ta, tb, row, col, cx, cy, cz, g_o, es_o,
                rowi, coli, bufa0, bufb0, bufa1, bufb1, gsum, esb,
                cxv, cyv, czv, sga0, sgb0, sga1, sgb1, semw0, semw1):
        wid = lax.axis_index("s") * NC + lax.axis_index("c")
        base = wid * epw
        pltpu.sync_copy(row.at[pl.ds(base, epw)], rowi)
        pltpu.sync_copy(col.at[pl.ds(base, epw)], coli)
        pltpu.sync_copy(cx, cxv)
        pltpu.sync_copy(cy, cyv)
        pltpu.sync_copy(cz, czv)
        iota = lax.iota(jnp.int32, 16)

        def issue_g(c, bufa, bufb, sa, sb):
            pltpu.async_copy(ta.at[rowi.at[pl.ds(c * KCH, KCH)]], bufa, sa)
            pltpu.async_copy(tb.at[coli.at[pl.ds(c * KCH, KCH)]], bufb, sb)

        def wait_g(c, bufa, bufb, sa, sb):
            pltpu.make_async_copy(ta.at[rowi.at[pl.ds(c * KCH, KCH)]],
                                  bufa, sa).wait()
            pltpu.make_async_copy(tb.at[coli.at[pl.ds(c * KCH, KCH)]],
                                  bufb, sb).wait()

        def compute_es(c):
            for j in range(KCH // 16):
                rv = rowi[pl.ds(c * KCH + j * 16, 16)]
                cv = coli[pl.ds(c * KCH + j * 16, 16)]
                dx = (plsc.load_gather(cxv, [rv])
                      - plsc.load_gather(cxv, [cv]))
                dy = (plsc.load_gather(cyv, [rv])
                      - plsc.load_gather(cyv, [cv]))
                dz = (plsc.load_gather(czv, [rv])
                      - plsc.load_gather(czv, [cv]))
                r = dx * dx + dy * dy + dz * dz
                ri = iota + (j * 16)
                plsc.store_scatter(esb, [ri, iota * 0], r)
                plsc.store_scatter(esb, [ri, iota * 0 + 1], dx)
                plsc.store_scatter(esb, [ri, iota * 0 + 2], dy)
                plsc.store_scatter(esb, [ri, iota * 0 + 3], dz)

        def add_rows(bufa, bufb):
            def radd(r, _):
                for j in range(HID // 16):
                    gsum[r, pl.ds(j * 16, 16)] = (
                        bufa[r, pl.ds(j * 16, 16)]
                        + bufb[r, pl.ds(j * 16, 16)])
                return 0

            lax.fori_loop(0, KCH, radd, 0)

        def drain(c, semw):
            off = base + c * KCH
            wg = pltpu.async_copy(gsum, g_o.at[pl.ds(off, KCH)], semw)
            we = pltpu.async_copy(esb, es_o.at[pl.ds(off, KCH)], semw)
            wg.wait()
            we.wait()

        def step(c, bufa, bufb, sa, sb, semw):
            compute_es(c)
            wait_g(c, bufa, bufb, sa, sb)
            add_rows(bufa, bufb)
            drain(c, semw)

        issue_g(0, bufa0, bufb0, sga0, sgb0)
        issue_g(1, bufa1, bufb1, sga1, sgb1)

        def body(i, _):
            c0 = 2 * i
            c1 = c0 + 1
            step(c0, bufa0, bufb0, sga0, sgb0, semw0)
            issue_g(c0 + 2, bufa0, bufb0, sga0, sgb0)
            step(c1, bufa1, bufb1, sga1, sgb1, semw1)
            if odd:
                @pl.when(i < pairs - 1)
                def _():
                    issue_g(c1 + 2, bufa1, bufb1, sga1, sgb1)
            else:
                issue_g(c1 + 2, bufa1, bufb1, sga1, sgb1)
            return 0

        lax.fori_loop(0, pairs, body, 0)
        if odd:
            step(nch - 1, bufa0, bufb0, sga0, sgb0, semw0)
        else:
            step(nch - 2, bufa0, bufb0, sga0, sgb0, semw0)
            step(nch - 1, bufa1, bufb1, sga1, sgb1, semw1)

    return body_fn


def _make_scatter_m_body(epw, nch):
    pairs = (nch - 1) // 2
    odd = nch % 2 == 1

    def body_fn(m, row3, zrows, pm_o, idr, mb0, mb1, accm,
                sl0, sl1, sa0, sa1):
        cid = lax.axis_index("c")
        sid = lax.axis_index("s")
        wid = sid * NC + cid
        base = wid * epw

        def issue_l(c, mb, sl):
            pltpu.async_copy(m.at[pl.ds(base + c * KCH, KCH)], mb, sl)

        def wait_l(c, mb, sl):
            pltpu.make_async_copy(m.at[pl.ds(base + c * KCH, KCH)], mb,
                                  sl).wait()

        issue_l(0, mb0, sl0)
        issue_l(1, mb1, sl1)
        pltpu.sync_copy(row3.at[wid], idr)
        pltpu.sync_copy(zrows, accm.at[pl.ds(sid * SPT, SPT)])
        plsc.subcore_barrier()

        def sadd(c, mb, sa):
            pltpu.async_copy(mb, accm.at[idr.at[c]], sa, add=True).wait()

        def step(c, mb, sl, sa):
            wait_l(c, mb, sl)
            sadd(c, mb, sa)

        def body(i, _):
            c0 = 2 * i
            c1 = c0 + 1
            step(c0, mb0, sl0, sa0)
            issue_l(c0 + 2, mb0, sl0)
            step(c1, mb1, sl1, sa1)
            if odd:
                @pl.when(i < pairs - 1)
                def _():
                    issue_l(c1 + 2, mb1, sl1)
            else:
                issue_l(c1 + 2, mb1, sl1)
            return 0

        lax.fori_loop(0, pairs, body, 0)
        if odd:
            step(nch - 1, mb0, sl0, sa0)
        else:
            step(nch - 2, mb0, sl0, sa0)
            step(nch - 1, mb1, sl1, sa1)
        plsc.subcore_barrier()
        pltpu.sync_copy(accm.at[pl.ds(sid * SPT, SPT)],
                        pm_o.at[cid, pl.ds(sid * SPT, SPT)])

    return body_fn


def _make_scatter_t_body(epw, nch):
    pairs = (nch - 1) // 2
    odd = nch % 2 == 1

    def body_fn(s16, row3, pt_o, idr, sb0, sb1, acct, ss0, ss1):
        cid = lax.axis_index("c")
        sid = lax.axis_index("s")
        wid = sid * NC + cid
        base = wid * epw

        def issue_l(c, sb, ss):
            pltpu.async_copy(s16.at[pl.ds(base + c * KCH, KCH)], sb, ss)

        def wait_l(c, sb, ss):
            pltpu.make_async_copy(s16.at[pl.ds(base + c * KCH, KCH)], sb,
                                  ss).wait()

        issue_l(0, sb0, ss0)
        issue_l(1, sb1, ss1)
        pltpu.sync_copy(row3.at[wid], idr)
        zero16 = jnp.zeros((16,), F32)

        def zbody(i, _):
            for j in range(HID // 16):
                acct[i, pl.ds(j * 16, 16)] = zero16
            return 0

        lax.fori_loop(0, TR, zbody, 0)
        iota = lax.iota(jnp.int32, 16)
        one16 = zero16 + 1.0

        def process(c, sb):
            for j in range(KCH // 16):
                rv = idr[c, pl.ds(j * 16, 16)]
                ri = iota + (j * 16)
                sv = plsc.load_gather(sb, [ri, iota * 0])
                dx = plsc.load_gather(sb, [ri, iota * 0 + 1])
                dy = plsc.load_gather(sb, [ri, iota * 0 + 2])
                dz = plsc.load_gather(sb, [ri, iota * 0 + 3])
                fb = rv * NT
                for k, v in ((0, dx * sv), (1, dy * sv), (2, dz * sv),
                             (3, one16)):
                    fk = fb + k
                    plsc.addupdate_scatter(
                        acct, [lax.shift_right_logical(fk, 7), fk & 127], v)

        def step(c, sb, ss):
            wait_l(c, sb, ss)
            process(c, sb)

        def body(i, _):
            c0 = 2 * i
            c1 = c0 + 1
            step(c0, sb0, ss0)
            issue_l(c0 + 2, sb0, ss0)
            step(c1, sb1, ss1)
            if odd:
                @pl.when(i < pairs - 1)
                def _():
                    issue_l(c1 + 2, sb1, ss1)
            else:
                issue_l(c1 + 2, sb1, ss1)
            return 0

        lax.fori_loop(0, pairs, body, 0)
        if odd:
            step(nch - 1, sb0, ss0)
        else:
            step(nch - 2, sb0, ss0)
            step(nch - 1, sb1, ss1)
        pltpu.sync_copy(acct, pt_o.at[wid])

    return body_fn


@functools.cache
def _sc_kernels(epw, nch):
    eh = epw * NW
    mesh = plsc.VectorSubcoreMesh(core_axis_name="c", subcore_axis_name="s",
                                  num_cores=NC, num_subcores=NS)
    cparams = pltpu.CompilerParams(needs_layout_passes=False)
    gather = functools.partial(
        pl.kernel,
        compiler_params=cparams,
        out_type=(jax.ShapeDtypeStruct((eh, HID), F32),
                  jax.ShapeDtypeStruct((eh, TAIL), F32)),
        mesh=mesh,
        scratch_types=[
            pltpu.VMEM((epw,), jnp.int32),
            pltpu.VMEM((epw,), jnp.int32),
            pltpu.VMEM((KCH, HID), F32),
            pltpu.VMEM((KCH, HID), F32),
            pltpu.VMEM((KCH, HID), F32),
            pltpu.VMEM((KCH, HID), F32),
            pltpu.VMEM((KCH, HID), F32),
            pltpu.VMEM((KCH, TAIL), F32),
            pltpu.VMEM((N,), F32),
            pltpu.VMEM((N,), F32),
            pltpu.VMEM((N,), F32),
            pltpu.SemaphoreType.DMA,
            pltpu.SemaphoreType.DMA,
            pltpu.SemaphoreType.DMA,
            pltpu.SemaphoreType.DMA,
            pltpu.SemaphoreType.DMA,
            pltpu.SemaphoreType.DMA,
        ],
    )(_make_gather_body(epw, nch))
    scatter_m = functools.partial(
        pl.kernel,
        compiler_params=cparams,
        out_type=jax.ShapeDtypeStruct((NC, NP, HID), F32),
        mesh=mesh,
        scratch_types=[
            pltpu.VMEM((nch, KCH), jnp.int32),
            pltpu.VMEM((KCH, HID), F32),
            pltpu.VMEM((KCH, HID), F32),
            pltpu.VMEM_SHARED((NP, HID), F32),
            pltpu.SemaphoreType.DMA,
            pltpu.SemaphoreType.DMA,
            pltpu.SemaphoreType.DMA,
            pltpu.SemaphoreType.DMA,
        ],
    )(_make_scatter_m_body(epw, nch))
    scatter_t = functools.partial(
        pl.kernel,
        compiler_params=cparams,
        out_type=jax.ShapeDtypeStruct((NW, TR, HID), F32),
        mesh=mesh,
        scratch_types=[
            pltpu.VMEM((nch, KCH), jnp.int32),
            pltpu.VMEM((KCH, TAIL), F32),
            pltpu.VMEM((KCH, TAIL), F32),
            pltpu.VMEM((TR, HID), F32),
            pltpu.SemaphoreType.DMA,
            pltpu.SemaphoreType.DMA,
        ],
    )(_make_scatter_t_body(epw, nch))
    return gather, scatter_m, scatter_t


NCHA, NCHB = 63, 62
EPWA, EPWB = NCHA * KCH, NCHB * KCH
EA_, EB_ = EPWA * NW, EPWB * NW


def _sc_gather(half, ta, tb, row, col, cx, cy, cz):
    epw, nch = (EPWA, NCHA) if half == 0 else (EPWB, NCHB)
    return _sc_kernels(epw, nch)[0](ta, tb, row, col, cx, cy, cz)


def _sc_scatter_m(half, m, row3, zrows):
    epw, nch = (EPWA, NCHA) if half == 0 else (EPWB, NCHB)
    return _sc_kernels(epw, nch)[1](m, row3, zrows)


def _sc_scatter_t(half, s16, row3):
    epw, nch = (EPWA, NCHA) if half == 0 else (EPWB, NCHB)
    return _sc_kernels(epw, nch)[2](s16, row3)


BN = 2000
BE = 2560


def _full(shape):
    return pl.BlockSpec(shape, lambda i: (0,) * len(shape))


def _blk(shape, pos=0):
    def imap(i):
        out = [0] * len(shape)
        out[pos] = i
        return tuple(out)
    return pl.BlockSpec(shape, imap)


def _tc_node0(h, cp, wemb, bemb, w1a, be1, w1b):
    return pl.pallas_call(
        _node0_body,
        grid=(N // BN,),
        in_specs=[_blk((BN, HID)), _blk((BN, TAIL)), _full((HID, HID)),
                  _full((1, HID)), _full((HID, HID)), _full((1, HID)),
                  _full((HID, HID))],
        out_specs=[_blk((BN, HID)), _blk((BN, HID)), _blk((BN, HID))],
        out_shape=[jax.ShapeDtypeStruct((N, HID), F32),
                   jax.ShapeDtypeStruct((N, HID), F32),
                   jax.ShapeDtypeStruct((N, HID), F32)],
    )(h, cp, wemb, bemb, w1a, be1, w1b)


def _tc_node(h, cp, vp, p0, p1, p2, p3, pt, wv1, bv1, wv2, bv2, wn1a, wn1b,
             bn1, wn2, bn2, w1a, be1, w1b):
    return pl.pallas_call(
        _node_body,
        grid=(N // BN,),
        in_specs=[_blk((BN, HID)), _blk((BN, TAIL)), _blk((BN, TAIL)),
                  _blk((BN, HID)), _blk((BN, HID)), _blk((BN, HID)),
                  _blk((BN, HID)), _blk((BN, NT)),
                  _full((HID, HID)), _full((1, HID)), _full((HID, 1)),
                  _full((1, 1)),
                  _full((HID, HID)), _full((HID, HID)), _full((1, HID)),
                  _full((HID, HID)), _full((1, HID)),
                  _full((HID, HID)), _full((1, HID)), _full((HID, HID))],
        out_specs=[_blk((BN, HID)), _blk((BN, TAIL)), _blk((BN, HID)),
                   _blk((BN, HID))],
        out_shape=[jax.ShapeDtypeStruct((N, HID), F32),
                   jax.ShapeDtypeStruct((N, TAIL), F32),
                   jax.ShapeDtypeStruct((N, HID), F32),
                   jax.ShapeDtypeStruct((N, HID), F32)],
    )(h, cp, vp, p0, p1, p2, p3, pt, wv1, bv1, wv2, bv2, wn1a, wn1b, bn1,
      wn2, bn2, w1a, be1, w1b)


def _tc_node_last(h, cp, vp, pt, wv1, bv1, wv2, bv2):
    return pl.pallas_call(
        _node_last_body,
        grid=(N // BN,),
        in_specs=[_blk((BN, HID)), _blk((BN, TAIL)), _blk((BN, TAIL)),
                  _blk((BN, NT)),
                  _full((HID, HID)), _full((1, HID)), _full((HID, 1)),
                  _full((1, 1))],
        out_specs=_blk((BN, TAIL)),
        out_shape=jax.ShapeDtypeStruct((N, TAIL), F32),
    )(h, cp, vp, pt, wv1, bv1, wv2, bv2)


def _tc_reduce(pta, ptb):
    return pl.pallas_call(
        _reduce_body,
        grid=(1,),
        in_specs=[_full((NW, TR, HID)), _full((NW, TR, HID))],
        out_specs=_full((TR, HID)),
        out_shape=jax.ShapeDtypeStruct((TR, HID), F32),
    )(pta, ptb)


def _tc_edge(g, es, ea, wr, w1e, we2, be2, wc1, bc1, wc2, want_m=True):
    eh = g.shape[0]
    body = _edge_body if want_m else _edge_body_nom
    out_specs = [_blk((BE, HID)), _blk((BE, TAIL))]
    out_shape = [jax.ShapeDtypeStruct((eh, HID), F32),
                 jax.ShapeDtypeStruct((eh, TAIL), F32)]
    if not want_m:
        out_specs, out_shape = out_specs[1:], out_shape[1:]
    res = pl.pallas_call(
        body,
        grid=(eh // BE,),
        in_specs=[_blk((BE, HID)), _blk((BE, TAIL)),
                  _blk((BE, 16)),
                  _full((1, HID)), _full((16, HID)), _full((HID, HID)),
                  _full((1, HID)), _full((HID, HID)), _full((1, HID)),
                  _full((HID, 1))],
        out_specs=out_specs,
        out_shape=out_shape,
    )(g, es, ea, wr, w1e, we2, be2, wc1, bc1, wc2)
    return res if want_m else res[0]


def kernel(h, x, edges, vel, edge_attr, params):
    row = edges[0]
    col = edges[1]
    rows = (row[:EA_], row[EA_:])
    cols = (col[:EA_], col[EA_:])
    row3s = (rows[0].reshape(NW, NCHA, KCH), rows[1].reshape(NW, NCHB, KCH))
    eas = (edge_attr[:EA_], edge_attr[EA_:])
    cp = jnp.pad(x, ((0, 0), (0, TAIL - 3)))
    vp = jnp.pad(vel, ((0, 0), (0, TAIL - 3)))
    zrows = jnp.zeros((SPT, HID), F32)
    r2 = lambda b: b.reshape(1, -1)

    lp = params["layers"]
    p0w = lp[0]
    hh, ta, tb = _tc_node0(
        h, cp, params["emb"]["W"], r2(params["emb"]["b"]),
        p0w["We1"][:HID], r2(p0w["be1"]), p0w["We1"][HID:2 * HID])

    for li in range(4):
        p = lp[li]
        ew = (r2(p["We1"][2 * HID]), p["We1"][2 * HID + 1:], p["We2"],
              r2(p["be2"]), p["Wc1"], r2(p["bc1"]), p["Wc2"])
        cx, cy, cz = cp[:, 0], cp[:, 1], cp[:, 2]
        last = li == 3
        g0, es0 = _sc_gather(0, ta, tb, rows[0], cols[0], cx, cy, cz)
        g1, es1 = _sc_gather(1, ta, tb, rows[1], cols[1], cx, cy, cz)
        if not last:
            m0, s0 = _tc_edge(g0, es0, eas[0], *ew)
            m1, s1 = _tc_edge(g1, es1, eas[1], *ew)
            pmA = _sc_scatter_m(0, m0, row3s[0], zrows)
            pmB = _sc_scatter_m(1, m1, row3s[1], zrows)
        else:
            s0 = _tc_edge(g0, es0, eas[0], *ew, want_m=False)
            s1 = _tc_edge(g1, es1, eas[1], *ew, want_m=False)
        ptA = _sc_scatter_t(0, s0, row3s[0])
        ptB = _sc_scatter_t(1, s1, row3s[1])
        ptr = _tc_reduce(ptA, ptB).reshape(TR * HID)[:N * NT].reshape(N, NT)
        if not last:
            nx = lp[li + 1]
            hh, cp, ta, tb = _tc_node(
                hh, cp, vp, pmA[0, :N], pmA[1, :N], pmB[0, :N], pmB[1, :N],
                ptr,
                p["Wv1"], r2(p["bv1"]), p["Wv2"], r2(p["bv2"]),
                p["Wn1"][:HID], p["Wn1"][HID:], r2(p["bn1"]),
                p["Wn2"], r2(p["bn2"]),
                nx["We1"][:HID], r2(nx["be1"]), nx["We1"][HID:2 * HID])
        else:
            cp = _tc_node_last(hh, cp, vp, ptr,
                               p["Wv1"], r2(p["bv1"]), p["Wv2"],
                               r2(p["bv2"]))
    return cp[:, :3]
